# Initial kernel scaffold; baseline (speedup 1.0000x reference)
import functools

import jax
import jax.numpy as jnp
from jax import lax
from jax.experimental import pallas as pl
from jax.experimental.pallas import tpu as pltpu
from jax.experimental.pallas import tpu_sc as plsc

N = 10000
NP = 10016
F_IN = 128
HID = 64
HEADS = 8
HH = HID * HEADS
E = 160000
NT = 32
DPT = 313
CAP = 163840
FB = 2048
BN = 2504

_SC_PARAMS = pltpu.CompilerParams(needs_layout_passes=False)
_SDS = jax.ShapeDtypeStruct


def _wid():
    return lax.axis_index("s") * 2 + lax.axis_index("c")


def _scan_body(src_hbm, dst_hbm, lsrc_hbm, ldl_hbm, cnts_hbm,
               sbuf, dbuf, stg_s, stg_d, cntv):
    t = _wid()
    lo = t * DPT
    lanes = lax.iota(jnp.int32, 16)

    def append16(svec, dlvec, mask, pos_fb):
        pos, fbase = pos_fb
        mi = mask.astype(jnp.int32)
        ofs = plsc.cumsum(mi) - 1
        cnt = jnp.sum(mi, axis=0)
        plsc.store_scatter(stg_s, [pos + ofs], svec, mask=mask)
        plsc.store_scatter(stg_d, [pos + ofs], dlvec, mask=mask)
        pos = pos + cnt
        do_f = pos >= FB

        @pl.when(do_f)
        def _():
            base = t * CAP + fbase
            pltpu.sync_copy(stg_s.at[pl.ds(0, FB)], lsrc_hbm.at[pl.ds(base, FB)])
            pltpu.sync_copy(stg_d.at[pl.ds(0, FB)], ldl_hbm.at[pl.ds(base, FB)])
            ts_ = stg_s[pl.ds(FB, 16)]
            td_ = stg_d[pl.ds(FB, 16)]
            stg_s[pl.ds(0, 16)] = ts_
            stg_d[pl.ds(0, 16)] = td_

        pos = jnp.where(do_f, pos - FB, pos)
        fbase = jnp.where(do_f, fbase + FB, fbase)
        return pos, fbase

    C = 2048

    def chunk_body(nloc, pos_fb):
        def grp(g, pf):
            s16 = sbuf[pl.ds(g * 16, 16)]
            d16 = dbuf[pl.ds(g * 16, 16)]
            dl = d16 - lo
            m = jnp.logical_and(dl >= 0, dl < DPT)
            return append16(s16, dl, m, pf)
        return lax.fori_loop(0, nloc // 16, grp, pos_fb)

    def full_chunk(c, pf):
        base = c * C
        pltpu.sync_copy(src_hbm.at[pl.ds(base, C)], sbuf)
        pltpu.sync_copy(dst_hbm.at[pl.ds(base, C)], dbuf)
        return chunk_body(C, pf)

    pf = lax.fori_loop(0, E // C, full_chunk, (jnp.int32(0), jnp.int32(0)))
    TAIL = E - (E // C) * C
    if TAIL:
        pltpu.sync_copy(src_hbm.at[pl.ds(E - TAIL, TAIL)], sbuf.at[pl.ds(0, TAIL)])
        pltpu.sync_copy(dst_hbm.at[pl.ds(E - TAIL, TAIL)], dbuf.at[pl.ds(0, TAIL)])
        pf = chunk_body(TAIL, pf)

    def selfloop(g, pf):
        dl = g * 16 + lanes
        dglob = lo + dl
        m = jnp.logical_and(dl < DPT, dglob < N)
        return append16(dglob, dl, m, pf)
    pf = lax.fori_loop(0, (DPT + 15) // 16, selfloop, pf)

    pos, fbase = pf
    total = pos + fbase
    target = jnp.bitwise_and(total + 127, jnp.int32(~127))
    k = target - total

    def padgrp(it, pf):
        m = (it * 16 + lanes) < k
        return append16(jnp.zeros((16,), jnp.int32),
                        jnp.full((16,), DPT, jnp.int32), m, pf)
    pos, fbase = lax.fori_loop(0, 8, padgrp, (pos, fbase))

    def fflush(kk, c):
        base = t * CAP + fbase + kk * 128
        pltpu.sync_copy(stg_s.at[pl.ds(kk * 128, 128)], lsrc_hbm.at[pl.ds(base, 128)])
        pltpu.sync_copy(stg_d.at[pl.ds(kk * 128, 128)], ldl_hbm.at[pl.ds(base, 128)])
        return c
    lax.fori_loop(0, pos // 128, fflush, 0)

    cntv[...] = jnp.zeros((16,), jnp.int32) + (fbase + pos)
    pltpu.sync_copy(cntv, cnts_hbm.at[t])


def _edge_scan(src, dst):
    mesh = plsc.VectorSubcoreMesh(core_axis_name="c", subcore_axis_name="s")
    scan = pl.kernel(
        _scan_body,
        out_type=(
            _SDS((NT * CAP,), jnp.int32),
            _SDS((NT * CAP,), jnp.int32),
            _SDS((NT, 16), jnp.int32),
        ),
        mesh=mesh,
        compiler_params=_SC_PARAMS,
        scratch_types=[
            pltpu.VMEM((2048,), jnp.int32),
            pltpu.VMEM((2048,), jnp.int32),
            pltpu.VMEM((FB + 32,), jnp.int32),
            pltpu.VMEM((FB + 32,), jnp.int32),
            pltpu.VMEM((16,), jnp.int32),
        ],
    )
    return scan(src, dst)


def _layer_body(hpA_hbm, hpB_hbm, sa_hbm, adstf_hbm, lsrc_hbm, ldl_hbm,
                cnts_hbm, outA_hbm, outB_hbm, exb_hbm,
                adst_own, den, cntv, sidx, dlv, srows, exstage,
                gi0, gi1, rows0, rows1, exv, dlh, acc, sem0, sem1):
    t = _wid()
    lo = t * DPT
    lanes = lax.iota(jnp.int32, 16)
    lane7 = jnp.bitwise_and(lanes, 7)
    mlo = lanes < 8
    lbase = t * CAP

    pltpu.sync_copy(cnts_hbm.at[t], cntv)
    npad = cntv[...][0]

    adst_own[pl.ds(DPT * 8 - 8, 16)] = jnp.zeros((16,), jnp.float32)
    pltpu.sync_copy(adstf_hbm.at[pl.ds(lo * 8, DPT * 8)], adst_own.at[pl.ds(0, DPT * 8)])

    def dzero(i, c):
        den[pl.ds(i * 16, 16)] = jnp.zeros((16,), jnp.float32)
        return c
    lax.fori_loop(0, (DPT + 1) * 8 // 16, dzero, 0)

    def p1_chunk(c, carry):
        base = c * 128
        pltpu.sync_copy(lsrc_hbm.at[pl.ds(lbase + base, 128)], sidx)
        pltpu.sync_copy(ldl_hbm.at[pl.ds(lbase + base, 128)], dlv)
        pltpu.async_copy(sa_hbm.at[sidx], srows, sem0).wait()

        def grp(g, c2):
            dl16 = dlv[pl.ds(g * 16, 16)]
            for j in range(16):
                e = g * 16 + j
                dl_s = dl16[j]
                srow = srows[e, pl.ds(0, 16)]
                aidx = dl_s * 8 + lane7
                adv = plsc.load_gather(adst_own, [aidx])
                s16 = srow + adv
                e16 = jnp.where(s16 > 0, s16, jnp.float32(0.2) * s16)
                ex = jnp.exp(e16)
                plsc.addupdate_scatter(den, [aidx], ex, mask=mlo)
                plsc.store_scatter(exstage, [e * 8 + lane7], ex, mask=mlo)
            return c2
        lax.fori_loop(0, 8, grp, 0)
        pltpu.sync_copy(exstage, exb_hbm.at[pl.ds((lbase + base) * 8, 1024)])
        return carry

    lax.fori_loop(0, npad // 128, p1_chunk, 0)

    def p2(i, c):
        v = den[pl.ds(i * 16, 16)]
        den[pl.ds(i * 16, 16)] = jnp.float32(1.0) / (v + jnp.float32(1e-16))
        return c
    lax.fori_loop(0, (DPT + 1) * 8 // 16, p2, 0)

    for r, (hp_hbm, out_hbm) in enumerate(((hpA_hbm, outA_hbm), (hpB_hbm, outB_hbm))):
        def azero(i, c):
            for u in range(4):
                acc[pl.ds(i * 64 + u * 16, 16)] = jnp.zeros((16,), jnp.float32)
            return c
        lax.fori_loop(0, (DPT + 1) * 256 // 64, azero, 0)

        pltpu.sync_copy(lsrc_hbm.at[pl.ds(lbase, 64)], gi0)
        pltpu.async_copy(hp_hbm.at[gi0], rows0, sem0).start()

        def process_half(eb, rows):
            pltpu.sync_copy(ldl_hbm.at[pl.ds(lbase + eb, 64)], dlh)
            pltpu.sync_copy(exb_hbm.at[pl.ds((lbase + eb) * 8, 512)], exv)

            def grp(g, c2):
                dl16 = dlh[pl.ds(g * 16, 16)]
                for j in range(16):
                    e = g * 16 + j
                    dl_s = dl16[j]
                    aidx = dl_s * 8 + lane7
                    rdv = plsc.load_gather(den, [aidx])
                    exe = plsc.load_gather(exv, [e * 8 + lane7])
                    alpha = exe * rdv
                    ab = dl_s * 256
                    for h in range(4):
                        a_b = jnp.zeros((16,), jnp.float32) + alpha[4 * r + h]
                        for q in range(4):
                            off = h * 64 + q * 16
                            rv = rows[e, pl.ds(off, 16)]
                            av = acc[pl.ds(ab + off, 16)]
                            acc[pl.ds(ab + off, 16)] = av + a_b * rv
                return c2
            lax.fori_loop(0, 4, grp, 0)

        def p3_iter(i, carry):
            base = i * 128
            pltpu.sync_copy(lsrc_hbm.at[pl.ds(lbase + base + 64, 64)], gi1)
            pltpu.async_copy(hp_hbm.at[gi1], rows1, sem1).start()
            pltpu.make_async_copy(hp_hbm.at[gi0], rows0, sem0).wait()
            process_half(base, rows0)
            @pl.when(base + 128 < npad)
            def _():
                pltpu.sync_copy(lsrc_hbm.at[pl.ds(lbase + base + 128, 64)], gi0)
                pltpu.async_copy(hp_hbm.at[gi0], rows0, sem0).start()
            pltpu.make_async_copy(hp_hbm.at[gi1], rows1, sem1).wait()
            process_half(base + 64, rows1)
            return carry

        lax.fori_loop(0, npad // 128, p3_iter, 0)
        pltpu.sync_copy(acc.at[pl.ds(0, DPT * 256)],
                        out_hbm.at[pl.ds(lo * 256, DPT * 256)])


def _gat_edge_phase(hpA, hpB, sa, adstf, lsrc, ldl, cnts):
    mesh = plsc.VectorSubcoreMesh(core_axis_name="c", subcore_axis_name="s")
    layer = pl.kernel(
        _layer_body,
        out_type=(
            _SDS((NP * 256,), jnp.float32),
            _SDS((NP * 256,), jnp.float32),
            _SDS((NT * CAP * 8,), jnp.float32),
        ),
        mesh=mesh,
        compiler_params=_SC_PARAMS,
        scratch_types=[
            pltpu.VMEM(((DPT + 1) * 8,), jnp.float32),
            pltpu.VMEM(((DPT + 1) * 8,), jnp.float32),
            pltpu.VMEM((16,), jnp.int32),
            pltpu.VMEM((128,), jnp.int32),
            pltpu.VMEM((128,), jnp.int32),
            pltpu.VMEM((128, 16), jnp.float32),
            pltpu.VMEM((1024,), jnp.float32),
            pltpu.VMEM((64,), jnp.int32),
            pltpu.VMEM((64,), jnp.int32),
            pltpu.VMEM((64, 256), jnp.float32),
            pltpu.VMEM((64, 256), jnp.float32),
            pltpu.VMEM((512,), jnp.float32),
            pltpu.VMEM((64,), jnp.int32),
            pltpu.VMEM(((DPT + 1) * 256,), jnp.float32),
            pltpu.SemaphoreType.DMA,
            pltpu.SemaphoreType.DMA,
        ],
    )
    outA, outB, _ = layer(hpA, hpB, sa, adstf, lsrc, ldl, cnts)
    return outA.reshape(NP, 256), outB.reshape(NP, 256)


def _kin_body(x_ref, win_ref, bin_ref, w0_ref, asad_ref, hpA_ref, hpB_ref, sa_ref):
    h = jnp.dot(x_ref[...], win_ref[...], preferred_element_type=jnp.float32)
    h = h + bin_ref[...]
    hp = jnp.dot(h, w0_ref[...], preferred_element_type=jnp.float32)
    sa = jnp.dot(hp, asad_ref[...], preferred_element_type=jnp.float32)
    hpA_ref[...] = hp[:, :256]
    hpB_ref[...] = hp[:, 256:]
    sa_ref[...] = sa


def _tc_in(x_p, W_in, b_in, W0, asad):
    return pl.pallas_call(
        _kin_body,
        grid=(NP // BN,),
        in_specs=[
            pl.BlockSpec((BN, F_IN), lambda i: (i, 0)),
            pl.BlockSpec((F_IN, HID), lambda i: (0, 0)),
            pl.BlockSpec((1, HID), lambda i: (0, 0)),
            pl.BlockSpec((HID, HH), lambda i: (0, 0)),
            pl.BlockSpec((HH, 16), lambda i: (0, 0)),
        ],
        out_specs=[
            pl.BlockSpec((BN, 256), lambda i: (i, 0)),
            pl.BlockSpec((BN, 256), lambda i: (i, 0)),
            pl.BlockSpec((BN, 16), lambda i: (i, 0)),
        ],
        out_shape=[
            _SDS((NP, 256), jnp.float32),
            _SDS((NP, 256), jnp.float32),
            _SDS((NP, 16), jnp.float32),
        ],
    )(x_p, W_in, b_in.reshape(1, HID), W0, asad)


def _kmid_body(pA_ref, pB_ref, bA_ref, bB_ref, wA_ref, wB_ref, asad_ref,
               hpA_ref, hpB_ref, sa_ref):
    vA = pA_ref[...] + bA_ref[...]
    vB = pB_ref[...] + bB_ref[...]
    aA = jnp.where(vA > 0, vA, jnp.expm1(vA))
    aB = jnp.where(vB > 0, vB, jnp.expm1(vB))
    hp = (jnp.dot(aA, wA_ref[...], preferred_element_type=jnp.float32)
          + jnp.dot(aB, wB_ref[...], preferred_element_type=jnp.float32))
    sa = jnp.dot(hp, asad_ref[...], preferred_element_type=jnp.float32)
    hpA_ref[...] = hp[:, :256]
    hpB_ref[...] = hp[:, 256:]
    sa_ref[...] = sa


def _tc_mid(prevA, prevB, bias, W, asad):
    return pl.pallas_call(
        _kmid_body,
        grid=(NP // BN,),
        in_specs=[
            pl.BlockSpec((BN, 256), lambda i: (i, 0)),
            pl.BlockSpec((BN, 256), lambda i: (i, 0)),
            pl.BlockSpec((1, 256), lambda i: (0, 0)),
            pl.BlockSpec((1, 256), lambda i: (0, 0)),
            pl.BlockSpec((256, HH), lambda i: (0, 0)),
            pl.BlockSpec((256, HH), lambda i: (0, 0)),
            pl.BlockSpec((HH, 16), lambda i: (0, 0)),
        ],
        out_specs=[
            pl.BlockSpec((BN, 256), lambda i: (i, 0)),
            pl.BlockSpec((BN, 256), lambda i: (i, 0)),
            pl.BlockSpec((BN, 16), lambda i: (i, 0)),
        ],
        out_shape=[
            _SDS((NP, 256), jnp.float32),
            _SDS((NP, 256), jnp.float32),
            _SDS((NP, 16), jnp.float32),
        ],
    )(prevA, prevB, bias[:256].reshape(1, 256), bias[256:].reshape(1, 256),
      W[:256], W[256:], asad)


def _kfin_body(oA_ref, oB_ref, ssum_ref, b2_ref, wo1_ref, bo1_ref, wo2_ref,
               bo2_ref, wo3_ref, bo3_ref, out_ref, acc):
    i = pl.program_id(0)
    s = jnp.dot(oA_ref[...] + oB_ref[...], ssum_ref[...],
                preferred_element_type=jnp.float32)
    gid = lax.broadcasted_iota(jnp.int32, (BN, 1), 0) + i * BN
    s = jnp.where(gid < N, s, jnp.float32(0.0))
    part = jnp.sum(s, axis=0, keepdims=True)

    @pl.when(i == 0)
    def _():
        acc[...] = part

    @pl.when(i > 0)
    def _():
        acc[...] = acc[...] + part

    @pl.when(i == NP // BN - 1)
    def _():
        pooled = acc[...] * jnp.float32(1.0 / (HEADS * N)) + b2_ref[...]
        z = jnp.dot(pooled, wo1_ref[...], preferred_element_type=jnp.float32) + bo1_ref[...]
        z = jnp.maximum(z, 0.0)
        z = jnp.dot(z, wo2_ref[...], preferred_element_type=jnp.float32) + bo2_ref[...]
        z = jnp.maximum(z, 0.0)
        z = jnp.dot(z, wo3_ref[...], preferred_element_type=jnp.float32) + bo3_ref[...]
        z = z - jnp.max(z, axis=1, keepdims=True)
        ez = jnp.exp(z)
        out_ref[...] = ez / jnp.sum(ez, axis=1, keepdims=True)


def _tc_fin(oA, oB, ssum, bias2, Wo1, bo1, Wo2, bo2, Wo3, bo3):
    return pl.pallas_call(
        _kfin_body,
        grid=(NP // BN,),
        in_specs=[
            pl.BlockSpec((BN, 256), lambda i: (i, 0)),
            pl.BlockSpec((BN, 256), lambda i: (i, 0)),
            pl.BlockSpec((256, HID), lambda i: (0, 0)),
            pl.BlockSpec((1, HID), lambda i: (0, 0)),
            pl.BlockSpec((HID, HID), lambda i: (0, 0)),
            pl.BlockSpec((1, HID), lambda i: (0, 0)),
            pl.BlockSpec((HID, HID // 2), lambda i: (0, 0)),
            pl.BlockSpec((1, HID // 2), lambda i: (0, 0)),
            pl.BlockSpec((HID // 2, 3), lambda i: (0, 0)),
            pl.BlockSpec((1, 3), lambda i: (0, 0)),
        ],
        out_specs=pl.BlockSpec((1, 3), lambda i: (0, 0)),
        out_shape=_SDS((1, 3), jnp.float32),
        scratch_shapes=[pltpu.VMEM((1, HID), jnp.float32)],
    )(oA, oB, ssum, bias2.reshape(1, HID), Wo1, bo1.reshape(1, HID),
      Wo2, bo2.reshape(1, HID // 2), Wo3, bo3.reshape(1, 3))


def _asad_mat(a_src, a_dst):
    eye = jnp.eye(HEADS, dtype=jnp.float32)
    s = (eye[:, None, :] * a_src[:, :, None]).reshape(HH, HEADS)
    d = (eye[:, None, :] * a_dst[:, :, None]).reshape(HH, HEADS)
    return jnp.concatenate([s, d], axis=1)


def kernel(x, edge_index, W_in, b_in, W0, as0, ad0, bias0, W1, as1, ad1,
           bias1, W2, as2, ad2, bias2, Wo1, bo1, Wo2, bo2, Wo3, bo3):
    x_p = jnp.pad(x, ((0, NP - N), (0, 0)))
    lsrc, ldl, cnts = _edge_scan(edge_index[0], edge_index[1])

    def edge_phase(hpA, hpB, sa):
        adstf = sa[:, 8:].reshape(-1)
        return _gat_edge_phase(hpA, hpB, sa, adstf, lsrc, ldl, cnts)

    hpA, hpB, sa = _tc_in(x_p, W_in, b_in, W0, _asad_mat(as0, ad0))
    oA, oB = edge_phase(hpA, hpB, sa)
    hpA, hpB, sa = _tc_mid(oA, oB, bias0, W1, _asad_mat(as1, ad1))
    oA, oB = edge_phase(hpA, hpB, sa)
    hpA, hpB, sa = _tc_mid(oA, oB, bias1, W2, _asad_mat(as2, ad2))
    oA, oB = edge_phase(hpA, hpB, sa)

    ssum = jnp.tile(jnp.eye(HID, dtype=jnp.float32), (4, 1))
    return _tc_fin(oA, oB, ssum, bias2, Wo1, bo1, Wo2, bo2, Wo3, bo3)


# speedup vs baseline: 7.4146x; 7.4146x over previous
import functools

import jax
import jax.numpy as jnp
from jax import lax
from jax.experimental import pallas as pl
from jax.experimental.pallas import tpu as pltpu
from jax.experimental.pallas import tpu_sc as plsc

N = 10000
NP = 10016
F_IN = 128
HID = 64
HEADS = 8
HH = HID * HEADS
E = 160000
NT = 32
DPT = 313
CAP = 163840
FB = 2048
BN = 2504

_SC_PARAMS = pltpu.CompilerParams(needs_layout_passes=False)
_SDS = jax.ShapeDtypeStruct


def _wid():
    return lax.axis_index("s") * 2 + lax.axis_index("c")


def _scan_body(src_hbm, dst_hbm, lsrc_hbm, ldl_hbm, cnts_hbm,
               sbuf, dbuf, stg_s, stg_d, cntv):
    t = _wid()
    lo = t * DPT
    lanes = lax.iota(jnp.int32, 16)

    def append16(svec, dlvec, mask, pos_fb):
        pos, fbase = pos_fb
        mi = mask.astype(jnp.int32)
        ofs = plsc.cumsum(mi) - 1
        cnt = jnp.sum(mi, axis=0)
        plsc.store_scatter(stg_s, [pos + ofs], svec, mask=mask)
        plsc.store_scatter(stg_d, [pos + ofs], dlvec, mask=mask)
        pos = pos + cnt
        do_f = pos >= FB

        @pl.when(do_f)
        def _():
            base = pl.multiple_of(t * CAP + fbase, FB)
            pltpu.sync_copy(stg_s.at[pl.ds(0, FB)], lsrc_hbm.at[pl.ds(base, FB)])
            pltpu.sync_copy(stg_d.at[pl.ds(0, FB)], ldl_hbm.at[pl.ds(base, FB)])
            ts_ = stg_s[pl.ds(FB, 16)]
            td_ = stg_d[pl.ds(FB, 16)]
            stg_s[pl.ds(0, 16)] = ts_
            stg_d[pl.ds(0, 16)] = td_

        pos = jnp.where(do_f, pos - FB, pos)
        fbase = jnp.where(do_f, fbase + FB, fbase)
        return pos, fbase

    C = 2048

    def chunk_body(nloc, pos_fb):
        def grp(g, pf):
            s16 = sbuf[pl.ds(g * 16, 16)]
            d16 = dbuf[pl.ds(g * 16, 16)]
            dl = d16 - lo
            m = jnp.logical_and(dl >= 0, dl < DPT)
            return append16(s16, dl, m, pf)
        return lax.fori_loop(0, nloc // 16, grp, pos_fb)

    def full_chunk(c, pf):
        base = pl.multiple_of(c * C, C)
        pltpu.sync_copy(src_hbm.at[pl.ds(base, C)], sbuf)
        pltpu.sync_copy(dst_hbm.at[pl.ds(base, C)], dbuf)
        return chunk_body(C, pf)

    pf = lax.fori_loop(0, E // C, full_chunk, (jnp.int32(0), jnp.int32(0)))
    TAIL = E - (E // C) * C
    if TAIL:
        pltpu.sync_copy(src_hbm.at[pl.ds(E - TAIL, TAIL)], sbuf.at[pl.ds(0, TAIL)])
        pltpu.sync_copy(dst_hbm.at[pl.ds(E - TAIL, TAIL)], dbuf.at[pl.ds(0, TAIL)])
        pf = chunk_body(TAIL, pf)

    def selfloop(g, pf):
        dl = g * 16 + lanes
        dglob = lo + dl
        m = jnp.logical_and(dl < DPT, dglob < N)
        return append16(dglob, dl, m, pf)
    pf = lax.fori_loop(0, (DPT + 15) // 16, selfloop, pf)

    pos, fbase = pf
    total = pos + fbase
    target = jnp.bitwise_and(total + 127, jnp.int32(~127))
    k = target - total

    def padgrp(it, pf):
        m = (it * 16 + lanes) < k
        return append16(jnp.zeros((16,), jnp.int32),
                        jnp.full((16,), DPT, jnp.int32), m, pf)
    pos, fbase = lax.fori_loop(0, 8, padgrp, (pos, fbase))

    def fflush(kk, c):
        base = pl.multiple_of(t * CAP + fbase + kk * 128, 128)
        pltpu.sync_copy(stg_s.at[pl.ds(kk * 128, 128)], lsrc_hbm.at[pl.ds(base, 128)])
        pltpu.sync_copy(stg_d.at[pl.ds(kk * 128, 128)], ldl_hbm.at[pl.ds(base, 128)])
        return c
    lax.fori_loop(0, pos // 128, fflush, 0)

    cntv[...] = jnp.zeros((16,), jnp.int32) + (fbase + pos)
    pltpu.sync_copy(cntv, cnts_hbm.at[t])


def _edge_scan(src, dst):
    mesh = plsc.VectorSubcoreMesh(core_axis_name="c", subcore_axis_name="s")
    scan = pl.kernel(
        _scan_body,
        out_type=(
            _SDS((NT * CAP,), jnp.int32),
            _SDS((NT * CAP,), jnp.int32),
            _SDS((NT, 16), jnp.int32),
        ),
        mesh=mesh,
        compiler_params=_SC_PARAMS,
        scratch_types=[
            pltpu.VMEM((2048,), jnp.int32),
            pltpu.VMEM((2048,), jnp.int32),
            pltpu.VMEM((FB + 32,), jnp.int32),
            pltpu.VMEM((FB + 32,), jnp.int32),
            pltpu.VMEM((16,), jnp.int32),
        ],
    )
    return scan(src, dst)


def _layer_body(hpA_hbm, hpB_hbm, sa_hbm, adstf_hbm, lsrc_hbm, ldl_hbm,
                cnts_hbm, outA_hbm, outB_hbm, exb_hbm,
                adst_own, den, cntv, sidx, dlv, srows, exstage,
                gi0, gi1, rows0, rows1, exv, dlh, acc, sem0, sem1):
    t = _wid()
    lo = t * DPT
    lanes = lax.iota(jnp.int32, 16)
    lane7 = jnp.bitwise_and(lanes, 7)
    mlo = lanes < 8
    lbase = t * CAP

    pltpu.sync_copy(cnts_hbm.at[t], cntv)
    npad = cntv[...][0]

    adst_own[pl.ds(DPT * 8 - 8, 16)] = jnp.zeros((16,), jnp.float32)
    pltpu.sync_copy(adstf_hbm.at[pl.ds(pl.multiple_of(lo * 8, 8), DPT * 8)], adst_own.at[pl.ds(0, DPT * 8)])

    def dzero(i, c):
        den[pl.ds(i * 16, 16)] = jnp.zeros((16,), jnp.float32)
        return c
    lax.fori_loop(0, (DPT + 1) * 8 // 16, dzero, 0)

    def p1_chunk(c, carry):
        base = pl.multiple_of(c * 64, 64)
        lb = pl.multiple_of(lbase + base, 64)
        pltpu.sync_copy(lsrc_hbm.at[pl.ds(lb, 64)], sidx)
        pltpu.sync_copy(ldl_hbm.at[pl.ds(lb, 64)], dlv)
        pltpu.async_copy(sa_hbm.at[sidx], srows, sem0).wait()

        def grp(g, c2):
            dl16 = dlv[pl.ds(pl.multiple_of(g * 16, 16), 16)]
            for j in range(16):
                e = g * 16 + j
                dl_s = dl16[j]
                srow = srows[e, pl.ds(0, 16)]
                aidx = dl_s * 8 + lane7
                adv = plsc.load_gather(adst_own, [aidx])
                s16 = srow + adv
                e16 = jnp.where(s16 > 0, s16, jnp.float32(0.2) * s16)
                ex = jnp.exp(e16)
                plsc.addupdate_scatter(den, [aidx], ex, mask=mlo)
                plsc.store_scatter(exstage, [e * 8 + lane7], ex, mask=mlo)
            return c2
        lax.fori_loop(0, 4, grp, 0)
        pltpu.sync_copy(exstage, exb_hbm.at[pl.ds(pl.multiple_of((lbase + base) * 8, 512), 512)])
        return carry

    lax.fori_loop(0, npad // 64, p1_chunk, 0)

    def p2(i, c):
        v = den[pl.ds(i * 16, 16)]
        den[pl.ds(i * 16, 16)] = jnp.float32(1.0) / (v + jnp.float32(1e-16))
        return c
    lax.fori_loop(0, (DPT + 1) * 8 // 16, p2, 0)

    for r, (hp_hbm, out_hbm) in enumerate(((hpA_hbm, outA_hbm), (hpB_hbm, outB_hbm))):
        def azero(i, c):
            for u in range(4):
                acc[pl.ds(i * 64 + u * 16, 16)] = jnp.zeros((16,), jnp.float32)
            return c
        lax.fori_loop(0, (DPT + 1) * 256 // 64, azero, 0)

        pltpu.sync_copy(lsrc_hbm.at[pl.ds(pl.multiple_of(lbase, 64), 64)], gi0)
        pltpu.make_async_copy(hp_hbm.at[gi0], rows0, sem0).start()

        def process_half(eb, rows):
            eb2 = pl.multiple_of(lbase + eb, 64)
            pltpu.sync_copy(ldl_hbm.at[pl.ds(eb2, 64)], dlh)
            pltpu.sync_copy(exb_hbm.at[pl.ds(pl.multiple_of((lbase + eb) * 8, 512), 512)], exv)

            def grp(g, c2):
                dl16 = dlh[pl.ds(pl.multiple_of(g * 16, 16), 16)]
                for j in range(16):
                    e = g * 16 + j
                    dl_s = dl16[j]
                    aidx = dl_s * 8 + lane7
                    rdv = plsc.load_gather(den, [aidx])
                    exe = plsc.load_gather(exv, [e * 8 + lane7])
                    alpha = exe * rdv
                    ab = pl.multiple_of(dl_s * 256, 256)
                    for h in range(4):
                        a_b = jnp.zeros((16,), jnp.float32) + alpha[4 * r + h]
                        for q in range(4):
                            off = h * 64 + q * 16
                            rv = rows[e, pl.ds(off, 16)]
                            av = acc[pl.ds(ab + off, 16)]
                            acc[pl.ds(ab + off, 16)] = av + a_b * rv
                return c2
            lax.fori_loop(0, 4, grp, 0)

        def p3_iter(i, carry):
            base = pl.multiple_of(i * 128, 128)
            pltpu.sync_copy(lsrc_hbm.at[pl.ds(pl.multiple_of(lbase + base + 64, 64), 64)], gi1)
            pltpu.make_async_copy(hp_hbm.at[gi1], rows1, sem1).start()
            pltpu.make_async_copy(hp_hbm.at[gi0], rows0, sem0).wait()
            process_half(base, rows0)
            @pl.when(base + 128 < npad)
            def _():
                pltpu.sync_copy(lsrc_hbm.at[pl.ds(pl.multiple_of(lbase + base + 128, 64), 64)], gi0)
                pltpu.make_async_copy(hp_hbm.at[gi0], rows0, sem0).start()
            pltpu.make_async_copy(hp_hbm.at[gi1], rows1, sem1).wait()
            process_half(base + 64, rows1)
            return carry

        lax.fori_loop(0, npad // 128, p3_iter, 0)
        pltpu.sync_copy(acc.at[pl.ds(0, DPT * 256)],
                        out_hbm.at[pl.ds(pl.multiple_of(lo * 256, 128), DPT * 256)])


def _gat_edge_phase(hpA, hpB, sa, adstf, lsrc, ldl, cnts):
    mesh = plsc.VectorSubcoreMesh(core_axis_name="c", subcore_axis_name="s")
    layer = pl.kernel(
        _layer_body,
        out_type=(
            _SDS((NP * 256,), jnp.float32),
            _SDS((NP * 256,), jnp.float32),
            _SDS((NT * CAP * 8,), jnp.float32),
        ),
        mesh=mesh,
        compiler_params=_SC_PARAMS,
        scratch_types=[
            pltpu.VMEM(((DPT + 1) * 8,), jnp.float32),
            pltpu.VMEM(((DPT + 1) * 8,), jnp.float32),
            pltpu.VMEM((16,), jnp.int32),
            pltpu.VMEM((64,), jnp.int32),
            pltpu.VMEM((64,), jnp.int32),
            pltpu.VMEM((64, 128), jnp.float32),
            pltpu.VMEM((512,), jnp.float32),
            pltpu.VMEM((64,), jnp.int32),
            pltpu.VMEM((64,), jnp.int32),
            pltpu.VMEM((64, 256), jnp.float32),
            pltpu.VMEM((64, 256), jnp.float32),
            pltpu.VMEM((512,), jnp.float32),
            pltpu.VMEM((64,), jnp.int32),
            pltpu.VMEM(((DPT + 1) * 256,), jnp.float32),
            pltpu.SemaphoreType.DMA,
            pltpu.SemaphoreType.DMA,
        ],
    )
    outA, outB, _ = layer(hpA, hpB, sa, adstf, lsrc, ldl, cnts)
    return outA.reshape(NP, 256), outB.reshape(NP, 256)


def _kin_body(x_ref, win_ref, bin_ref, w0_ref, asad_ref, hpA_ref, hpB_ref, sa_ref):
    h = jnp.dot(x_ref[...], win_ref[...], preferred_element_type=jnp.float32)
    h = h + bin_ref[...]
    hp = jnp.dot(h, w0_ref[...], preferred_element_type=jnp.float32)
    sa = jnp.dot(hp, asad_ref[...], preferred_element_type=jnp.float32,
                 precision=lax.Precision.HIGHEST)
    hpA_ref[...] = hp[:, :256]
    hpB_ref[...] = hp[:, 256:]
    sa_ref[...] = jnp.concatenate(
        [sa, jnp.zeros((sa.shape[0], 112), jnp.float32)], axis=1)


def _tc_in(x_p, W_in, b_in, W0, asad):
    return pl.pallas_call(
        _kin_body,
        grid=(NP // BN,),
        in_specs=[
            pl.BlockSpec((BN, F_IN), lambda i: (i, 0)),
            pl.BlockSpec((F_IN, HID), lambda i: (0, 0)),
            pl.BlockSpec((1, HID), lambda i: (0, 0)),
            pl.BlockSpec((HID, HH), lambda i: (0, 0)),
            pl.BlockSpec((HH, 16), lambda i: (0, 0)),
        ],
        out_specs=[
            pl.BlockSpec((BN, 256), lambda i: (i, 0)),
            pl.BlockSpec((BN, 256), lambda i: (i, 0)),
            pl.BlockSpec((BN, 128), lambda i: (i, 0)),
        ],
        out_shape=[
            _SDS((NP, 256), jnp.float32),
            _SDS((NP, 256), jnp.float32),
            _SDS((NP, 128), jnp.float32),
        ],
    )(x_p, W_in, b_in.reshape(1, HID), W0, asad)


def _kmid_body(pA_ref, pB_ref, bA_ref, bB_ref, w_ref, asad_ref,
               hpA_ref, hpB_ref, sa_ref):
    vA = pA_ref[...] + bA_ref[...]
    vB = pB_ref[...] + bB_ref[...]
    aA = jnp.where(vA > 0, vA, jnp.exp(jnp.minimum(vA, 0.0)) - 1.0)
    aB = jnp.where(vB > 0, vB, jnp.exp(jnp.minimum(vB, 0.0)) - 1.0)
    act = jnp.concatenate([aA, aB], axis=1)
    hp = jnp.dot(act, w_ref[...], preferred_element_type=jnp.float32)
    sa = jnp.dot(hp, asad_ref[...], preferred_element_type=jnp.float32,
                 precision=lax.Precision.HIGHEST)
    hpA_ref[...] = hp[:, :256]
    hpB_ref[...] = hp[:, 256:]
    sa_ref[...] = jnp.concatenate(
        [sa, jnp.zeros((sa.shape[0], 112), jnp.float32)], axis=1)


def _tc_mid(prevA, prevB, bias, W, asad):
    return pl.pallas_call(
        _kmid_body,
        grid=(NP // BN,),
        in_specs=[
            pl.BlockSpec((BN, 256), lambda i: (i, 0)),
            pl.BlockSpec((BN, 256), lambda i: (i, 0)),
            pl.BlockSpec((1, 256), lambda i: (0, 0)),
            pl.BlockSpec((1, 256), lambda i: (0, 0)),
            pl.BlockSpec((HH, HH), lambda i: (0, 0)),
            pl.BlockSpec((HH, 16), lambda i: (0, 0)),
        ],
        out_specs=[
            pl.BlockSpec((BN, 256), lambda i: (i, 0)),
            pl.BlockSpec((BN, 256), lambda i: (i, 0)),
            pl.BlockSpec((BN, 128), lambda i: (i, 0)),
        ],
        out_shape=[
            _SDS((NP, 256), jnp.float32),
            _SDS((NP, 256), jnp.float32),
            _SDS((NP, 128), jnp.float32),
        ],
    )(prevA, prevB, bias[:256].reshape(1, 256), bias[256:].reshape(1, 256),
      W, asad)


def _kfin_body(oA_ref, oB_ref, ssum_ref, b2_ref, wo1_ref, bo1_ref, wo2_ref,
               bo2_ref, wo3_ref, bo3_ref, out_ref, acc):
    i = pl.program_id(0)
    s = jnp.dot(oA_ref[...] + oB_ref[...], ssum_ref[...],
                preferred_element_type=jnp.float32,
                precision=lax.Precision.HIGHEST)
    gid = lax.broadcasted_iota(jnp.int32, (BN, 1), 0) + i * BN
    s = jnp.where(gid < N, s, jnp.float32(0.0))
    part = jnp.sum(s, axis=0, keepdims=True)

    @pl.when(i == 0)
    def _():
        acc[...] = part

    @pl.when(i > 0)
    def _():
        acc[...] = acc[...] + part

    @pl.when(i == NP // BN - 1)
    def _():
        pooled = acc[...] * jnp.float32(1.0 / (HEADS * N)) + b2_ref[...]
        z = jnp.dot(pooled, wo1_ref[...], preferred_element_type=jnp.float32) + bo1_ref[...]
        z = jnp.maximum(z, 0.0)
        z = jnp.dot(z, wo2_ref[...], preferred_element_type=jnp.float32) + bo2_ref[...]
        z = jnp.maximum(z, 0.0)
        z = jnp.dot(z, wo3_ref[...], preferred_element_type=jnp.float32) + bo3_ref[...]
        z = z - jnp.max(z, axis=1, keepdims=True)
        ez = jnp.exp(z)
        out_ref[...] = ez / jnp.sum(ez, axis=1, keepdims=True)


def _tc_fin(oA, oB, ssum, bias2, Wo1, bo1, Wo2, bo2, Wo3, bo3):
    return pl.pallas_call(
        _kfin_body,
        grid=(NP // BN,),
        in_specs=[
            pl.BlockSpec((BN, 256), lambda i: (i, 0)),
            pl.BlockSpec((BN, 256), lambda i: (i, 0)),
            pl.BlockSpec((256, HID), lambda i: (0, 0)),
            pl.BlockSpec((1, HID), lambda i: (0, 0)),
            pl.BlockSpec((HID, HID), lambda i: (0, 0)),
            pl.BlockSpec((1, HID), lambda i: (0, 0)),
            pl.BlockSpec((HID, HID // 2), lambda i: (0, 0)),
            pl.BlockSpec((1, HID // 2), lambda i: (0, 0)),
            pl.BlockSpec((HID // 2, 3), lambda i: (0, 0)),
            pl.BlockSpec((1, 3), lambda i: (0, 0)),
        ],
        out_specs=pl.BlockSpec((1, 3), lambda i: (0, 0)),
        out_shape=_SDS((1, 3), jnp.float32),
        scratch_shapes=[pltpu.VMEM((1, HID), jnp.float32)],
    )(oA, oB, ssum, bias2.reshape(1, HID), Wo1, bo1.reshape(1, HID),
      Wo2, bo2.reshape(1, HID // 2), Wo3, bo3.reshape(1, 3))


def _asad_mat(a_src, a_dst):
    eye = jnp.eye(HEADS, dtype=jnp.float32)
    s = (eye[:, None, :] * a_src[:, :, None]).reshape(HH, HEADS)
    d = (eye[:, None, :] * a_dst[:, :, None]).reshape(HH, HEADS)
    return jnp.concatenate([s, d], axis=1)


def kernel(x, edge_index, W_in, b_in, W0, as0, ad0, bias0, W1, as1, ad1,
           bias1, W2, as2, ad2, bias2, Wo1, bo1, Wo2, bo2, Wo3, bo3):
    x_p = jnp.pad(x, ((0, NP - N), (0, 0)))
    lsrc, ldl, cnts = _edge_scan(edge_index[0], edge_index[1])

    def edge_phase(hpA, hpB, sa):
        adstf = lax.optimization_barrier(sa[:, 8:16].reshape(-1))
        oA, oB = _gat_edge_phase(hpA, hpB, sa, adstf, lsrc, ldl, cnts)
        return lax.optimization_barrier(oA), lax.optimization_barrier(oB)

    hpA, hpB, sa = _tc_in(x_p, W_in, b_in, W0, _asad_mat(as0, ad0))
    oA, oB = edge_phase(hpA, hpB, sa)
    hpA, hpB, sa = _tc_mid(oA, oB, bias0, W1, _asad_mat(as1, ad1))
    oA, oB = edge_phase(hpA, hpB, sa)
    hpA, hpB, sa = _tc_mid(oA, oB, bias1, W2, _asad_mat(as2, ad2))
    oA, oB = edge_phase(hpA, hpB, sa)

    ssum = jnp.tile(jnp.eye(HID, dtype=jnp.float32), (4, 1))
    return _tc_fin(oA, oB, ssum, bias2, Wo1, bo1, Wo2, bo2, Wo3, bo3)


# speedup vs baseline: 12.1285x; 1.6358x over previous
import functools

import jax
import jax.numpy as jnp
from jax import lax
from jax.experimental import pallas as pl
from jax.experimental.pallas import tpu as pltpu
from jax.experimental.pallas import tpu_sc as plsc

N = 10000
NP = 10016
F_IN = 128
HID = 64
HEADS = 8
HH = HID * HEADS
E = 160000
NT = 32
DPT = 313
CAP = 163840
FB = 2048
BN = 2504

_SC_PARAMS = pltpu.CompilerParams(needs_layout_passes=False)
_SDS = jax.ShapeDtypeStruct


def _wid():
    return lax.axis_index("s") * 2 + lax.axis_index("c")


def _scan_body(src_hbm, dst_hbm, lsrc_hbm, ldl_hbm, cnts_hbm,
               sbuf, dbuf, stg_s, stg_d, cntv):
    t = _wid()
    lo = t * DPT
    lanes = lax.iota(jnp.int32, 16)

    def append16(svec, dlvec, mask, pos_fb):
        pos, fbase = pos_fb
        mi = mask.astype(jnp.int32)
        ofs = plsc.cumsum(mi) - 1
        cnt = jnp.sum(mi, axis=0)
        plsc.store_scatter(stg_s, [pos + ofs], svec, mask=mask)
        plsc.store_scatter(stg_d, [pos + ofs], dlvec, mask=mask)
        pos = pos + cnt
        do_f = pos >= FB

        @pl.when(do_f)
        def _():
            base = pl.multiple_of(t * CAP + fbase, FB)
            pltpu.sync_copy(stg_s.at[pl.ds(0, FB)], lsrc_hbm.at[pl.ds(base, FB)])
            pltpu.sync_copy(stg_d.at[pl.ds(0, FB)], ldl_hbm.at[pl.ds(base, FB)])
            ts_ = stg_s[pl.ds(FB, 16)]
            td_ = stg_d[pl.ds(FB, 16)]
            stg_s[pl.ds(0, 16)] = ts_
            stg_d[pl.ds(0, 16)] = td_

        pos = jnp.where(do_f, pos - FB, pos)
        fbase = jnp.where(do_f, fbase + FB, fbase)
        return pos, fbase

    C = 2048

    def chunk_body(nloc, pos_fb):
        def grp(g, pf):
            s16 = sbuf[pl.ds(g * 16, 16)]
            d16 = dbuf[pl.ds(g * 16, 16)]
            dl = d16 - lo
            m = jnp.logical_and(dl >= 0, dl < DPT)
            return append16(s16, dl, m, pf)
        return lax.fori_loop(0, nloc // 16, grp, pos_fb)

    def full_chunk(c, pf):
        base = pl.multiple_of(c * C, C)
        pltpu.sync_copy(src_hbm.at[pl.ds(base, C)], sbuf)
        pltpu.sync_copy(dst_hbm.at[pl.ds(base, C)], dbuf)
        return chunk_body(C, pf)

    pf = lax.fori_loop(0, E // C, full_chunk, (jnp.int32(0), jnp.int32(0)))
    TAIL = E - (E // C) * C
    if TAIL:
        pltpu.sync_copy(src_hbm.at[pl.ds(E - TAIL, TAIL)], sbuf.at[pl.ds(0, TAIL)])
        pltpu.sync_copy(dst_hbm.at[pl.ds(E - TAIL, TAIL)], dbuf.at[pl.ds(0, TAIL)])
        pf = chunk_body(TAIL, pf)

    def selfloop(g, pf):
        dl = g * 16 + lanes
        dglob = lo + dl
        m = jnp.logical_and(dl < DPT, dglob < N)
        return append16(dglob, dl, m, pf)
    pf = lax.fori_loop(0, (DPT + 15) // 16, selfloop, pf)

    pos, fbase = pf
    total = pos + fbase
    target = jnp.bitwise_and(total + 127, jnp.int32(~127))
    k = target - total

    def padgrp(it, pf):
        m = (it * 16 + lanes) < k
        return append16(jnp.zeros((16,), jnp.int32),
                        jnp.full((16,), DPT, jnp.int32), m, pf)
    pos, fbase = lax.fori_loop(0, 8, padgrp, (pos, fbase))

    def fflush(kk, c):
        base = pl.multiple_of(t * CAP + fbase + kk * 128, 128)
        pltpu.sync_copy(stg_s.at[pl.ds(kk * 128, 128)], lsrc_hbm.at[pl.ds(base, 128)])
        pltpu.sync_copy(stg_d.at[pl.ds(kk * 128, 128)], ldl_hbm.at[pl.ds(base, 128)])
        return c
    lax.fori_loop(0, pos // 128, fflush, 0)

    cntv[...] = jnp.zeros((16,), jnp.int32) + (fbase + pos)
    pltpu.sync_copy(cntv, cnts_hbm.at[t])


def _edge_scan(src, dst):
    mesh = plsc.VectorSubcoreMesh(core_axis_name="c", subcore_axis_name="s")
    scan = pl.kernel(
        _scan_body,
        out_type=(
            _SDS((NT * CAP,), jnp.int32),
            _SDS((NT * CAP,), jnp.int32),
            _SDS((NT, 16), jnp.int32),
        ),
        mesh=mesh,
        compiler_params=_SC_PARAMS,
        scratch_types=[
            pltpu.VMEM((2048,), jnp.int32),
            pltpu.VMEM((2048,), jnp.int32),
            pltpu.VMEM((FB + 32,), jnp.int32),
            pltpu.VMEM((FB + 32,), jnp.int32),
            pltpu.VMEM((16,), jnp.int32),
        ],
    )
    return scan(src, dst)


def _layer_body(hpA_hbm, hpB_hbm, sa_hbm, adstf_hbm, lsrc_hbm, ldl_hbm,
                cnts_hbm, outA_hbm, outB_hbm, exb_hbm,
                adst_own, den, cntv, sidx, dlv, srows, exstage,
                gi0, gi1, rows0, rows1, exv, dlh, acc, sem0, sem1):
    t = _wid()
    lo = t * DPT
    lanes = lax.iota(jnp.int32, 16)
    lane7 = jnp.bitwise_and(lanes, 7)
    mlo = lanes < 8
    lbase = t * CAP

    pltpu.sync_copy(cnts_hbm.at[t], cntv)
    npad = cntv[...][0]

    adst_own[pl.ds(DPT * 8 - 8, 16)] = jnp.zeros((16,), jnp.float32)
    pltpu.sync_copy(adstf_hbm.at[pl.ds(pl.multiple_of(lo * 8, 8), DPT * 8)], adst_own.at[pl.ds(0, DPT * 8)])

    def dzero(i, c):
        den[pl.ds(i * 16, 16)] = jnp.zeros((16,), jnp.float32)
        return c
    lax.fori_loop(0, (DPT + 1) * 8 // 16, dzero, 0)

    def p1_chunk(c, carry):
        base = pl.multiple_of(c * 64, 64)
        lb = pl.multiple_of(lbase + base, 64)
        pltpu.sync_copy(lsrc_hbm.at[pl.ds(lb, 64)], sidx)
        pltpu.sync_copy(ldl_hbm.at[pl.ds(lb, 64)], dlv)
        pltpu.async_copy(sa_hbm.at[sidx], srows, sem0).wait()

        def edge1(e, c2):
            es = jnp.zeros((16,), jnp.int32) + e
            srow = srows[e, pl.ds(0, 16)]
            dsp = plsc.load_gather(dlv, [es])
            aidx = dsp * 8 + lane7
            adv = plsc.load_gather(adst_own, [aidx])
            s16 = srow + adv
            e16 = jnp.where(s16 > 0, s16, jnp.float32(0.2) * s16)
            ex = jnp.exp(e16)
            plsc.addupdate_scatter(den, [aidx], ex, mask=mlo)
            plsc.store_scatter(exstage, [es * 8 + lane7], ex, mask=mlo)
            return c2
        lax.fori_loop(0, 64, edge1, 0)
        pltpu.sync_copy(exstage, exb_hbm.at[pl.ds(pl.multiple_of((lbase + base) * 8, 512), 512)])
        return carry

    lax.fori_loop(0, npad // 64, p1_chunk, 0)

    def p2(i, c):
        v = den[pl.ds(i * 16, 16)]
        den[pl.ds(i * 16, 16)] = jnp.float32(1.0) / (v + jnp.float32(1e-16))
        return c
    lax.fori_loop(0, (DPT + 1) * 8 // 16, p2, 0)

    for r, (hp_hbm, out_hbm) in enumerate(((hpA_hbm, outA_hbm), (hpB_hbm, outB_hbm))):
        def azero(i, c):
            for u in range(4):
                acc[pl.ds(i * 64 + u * 16, 16)] = jnp.zeros((16,), jnp.float32)
            return c
        lax.fori_loop(0, (DPT + 1) * 256 // 64, azero, 0)

        pltpu.sync_copy(lsrc_hbm.at[pl.ds(pl.multiple_of(lbase, 64), 64)], gi0)
        pltpu.make_async_copy(hp_hbm.at[gi0], rows0, sem0).start()

        def process_half(eb, rows):
            eb2 = pl.multiple_of(lbase + eb, 64)
            pltpu.sync_copy(ldl_hbm.at[pl.ds(eb2, 64)], dlh)
            pltpu.sync_copy(exb_hbm.at[pl.ds(pl.multiple_of((lbase + eb) * 8, 512), 512)], exv)

            def edge3(e, c2):
                es = jnp.zeros((16,), jnp.int32) + e
                dsp = plsc.load_gather(dlh, [es])
                exe = plsc.load_gather(exv, [es * 8 + lane7])
                rdv = plsc.load_gather(den, [dsp * 8 + lane7])
                alpha = exe * rdv
                abase = dsp * 256
                for v in range(16):
                    col = lanes + v * 16
                    rv = plsc.load_gather(rows, [es, col])
                    plsc.addupdate_scatter(acc, [abase + col], alpha * rv)
                return c2
            lax.fori_loop(0, 64, edge3, 0)

        def p3_iter(i, carry):
            base = pl.multiple_of(i * 128, 128)
            pltpu.sync_copy(lsrc_hbm.at[pl.ds(pl.multiple_of(lbase + base + 64, 64), 64)], gi1)
            pltpu.make_async_copy(hp_hbm.at[gi1], rows1, sem1).start()
            pltpu.make_async_copy(hp_hbm.at[gi0], rows0, sem0).wait()
            process_half(base, rows0)
            @pl.when(base + 128 < npad)
            def _():
                pltpu.sync_copy(lsrc_hbm.at[pl.ds(pl.multiple_of(lbase + base + 128, 64), 64)], gi0)
                pltpu.make_async_copy(hp_hbm.at[gi0], rows0, sem0).start()
            pltpu.make_async_copy(hp_hbm.at[gi1], rows1, sem1).wait()
            process_half(base + 64, rows1)
            return carry

        lax.fori_loop(0, npad // 128, p3_iter, 0)
        pltpu.sync_copy(acc.at[pl.ds(0, DPT * 256)],
                        out_hbm.at[pl.ds(pl.multiple_of(lo * 256, 128), DPT * 256)])


def _gat_edge_phase(hpA, hpB, sa, adstf, lsrc, ldl, cnts):
    mesh = plsc.VectorSubcoreMesh(core_axis_name="c", subcore_axis_name="s")
    layer = pl.kernel(
        _layer_body,
        out_type=(
            _SDS((NP * 256,), jnp.float32),
            _SDS((NP * 256,), jnp.float32),
            _SDS((NT * CAP * 8,), jnp.float32),
        ),
        mesh=mesh,
        compiler_params=_SC_PARAMS,
        scratch_types=[
            pltpu.VMEM(((DPT + 1) * 8,), jnp.float32),
            pltpu.VMEM(((DPT + 1) * 8,), jnp.float32),
            pltpu.VMEM((16,), jnp.int32),
            pltpu.VMEM((64,), jnp.int32),
            pltpu.VMEM((64,), jnp.int32),
            pltpu.VMEM((64, 128), jnp.float32),
            pltpu.VMEM((512,), jnp.float32),
            pltpu.VMEM((64,), jnp.int32),
            pltpu.VMEM((64,), jnp.int32),
            pltpu.VMEM((64, 256), jnp.float32),
            pltpu.VMEM((64, 256), jnp.float32),
            pltpu.VMEM((512,), jnp.float32),
            pltpu.VMEM((64,), jnp.int32),
            pltpu.VMEM(((DPT + 1) * 256,), jnp.float32),
            pltpu.SemaphoreType.DMA,
            pltpu.SemaphoreType.DMA,
        ],
    )
    outA, outB, _ = layer(hpA, hpB, sa, adstf, lsrc, ldl, cnts)
    return outA.reshape(NP, 256), outB.reshape(NP, 256)


def _kin_body(x_ref, win_ref, bin_ref, w0_ref, asad_ref, hpA_ref, hpB_ref, sa_ref):
    h = jnp.dot(x_ref[...], win_ref[...], preferred_element_type=jnp.float32)
    h = h + bin_ref[...]
    hp = jnp.dot(h, w0_ref[...], preferred_element_type=jnp.float32)
    sa = jnp.dot(hp, asad_ref[...], preferred_element_type=jnp.float32,
                 precision=lax.Precision.HIGHEST)
    hpA_ref[...] = hp[:, :256]
    hpB_ref[...] = hp[:, 256:]
    sa_ref[...] = jnp.concatenate(
        [sa, jnp.zeros((sa.shape[0], 112), jnp.float32)], axis=1)


def _tc_in(x_p, W_in, b_in, W0, asad):
    return pl.pallas_call(
        _kin_body,
        grid=(NP // BN,),
        in_specs=[
            pl.BlockSpec((BN, F_IN), lambda i: (i, 0)),
            pl.BlockSpec((F_IN, HID), lambda i: (0, 0)),
            pl.BlockSpec((1, HID), lambda i: (0, 0)),
            pl.BlockSpec((HID, HH), lambda i: (0, 0)),
            pl.BlockSpec((HH, 16), lambda i: (0, 0)),
        ],
        out_specs=[
            pl.BlockSpec((BN, 256), lambda i: (i, 0)),
            pl.BlockSpec((BN, 256), lambda i: (i, 0)),
            pl.BlockSpec((BN, 128), lambda i: (i, 0)),
        ],
        out_shape=[
            _SDS((NP, 256), jnp.float32),
            _SDS((NP, 256), jnp.float32),
            _SDS((NP, 128), jnp.float32),
        ],
    )(x_p, W_in, b_in.reshape(1, HID), W0, asad)


def _kmid_body(pA_ref, pB_ref, bA_ref, bB_ref, w_ref, asad_ref,
               hpA_ref, hpB_ref, sa_ref):
    vA = pA_ref[...] + bA_ref[...]
    vB = pB_ref[...] + bB_ref[...]
    aA = jnp.where(vA > 0, vA, jnp.exp(jnp.minimum(vA, 0.0)) - 1.0)
    aB = jnp.where(vB > 0, vB, jnp.exp(jnp.minimum(vB, 0.0)) - 1.0)
    act = jnp.concatenate([aA, aB], axis=1)
    hp = jnp.dot(act, w_ref[...], preferred_element_type=jnp.float32)
    sa = jnp.dot(hp, asad_ref[...], preferred_element_type=jnp.float32,
                 precision=lax.Precision.HIGHEST)
    hpA_ref[...] = hp[:, :256]
    hpB_ref[...] = hp[:, 256:]
    sa_ref[...] = jnp.concatenate(
        [sa, jnp.zeros((sa.shape[0], 112), jnp.float32)], axis=1)


def _tc_mid(prevA, prevB, bias, W, asad):
    return pl.pallas_call(
        _kmid_body,
        grid=(NP // BN,),
        in_specs=[
            pl.BlockSpec((BN, 256), lambda i: (i, 0)),
            pl.BlockSpec((BN, 256), lambda i: (i, 0)),
            pl.BlockSpec((1, 256), lambda i: (0, 0)),
            pl.BlockSpec((1, 256), lambda i: (0, 0)),
            pl.BlockSpec((HH, HH), lambda i: (0, 0)),
            pl.BlockSpec((HH, 16), lambda i: (0, 0)),
        ],
        out_specs=[
            pl.BlockSpec((BN, 256), lambda i: (i, 0)),
            pl.BlockSpec((BN, 256), lambda i: (i, 0)),
            pl.BlockSpec((BN, 128), lambda i: (i, 0)),
        ],
        out_shape=[
            _SDS((NP, 256), jnp.float32),
            _SDS((NP, 256), jnp.float32),
            _SDS((NP, 128), jnp.float32),
        ],
    )(prevA, prevB, bias[:256].reshape(1, 256), bias[256:].reshape(1, 256),
      W, asad)


def _kfin_body(oA_ref, oB_ref, ssA_ref, ssB_ref, b2_ref, wo1_ref, bo1_ref,
               wo2_ref, bo2_ref, wo3_ref, bo3_ref, out_ref, acc):
    i = pl.program_id(0)
    s = (jnp.dot(oA_ref[...], ssA_ref[...], preferred_element_type=jnp.float32,
                 precision=lax.Precision.HIGHEST)
         + jnp.dot(oB_ref[...], ssB_ref[...], preferred_element_type=jnp.float32,
                   precision=lax.Precision.HIGHEST))
    gid = lax.broadcasted_iota(jnp.int32, (BN, 1), 0) + i * BN
    s = jnp.where(gid < N, s, jnp.float32(0.0))
    part = jnp.sum(s, axis=0, keepdims=True)

    @pl.when(i == 0)
    def _():
        acc[...] = part

    @pl.when(i > 0)
    def _():
        acc[...] = acc[...] + part

    @pl.when(i == NP // BN - 1)
    def _():
        pooled = acc[...] * jnp.float32(1.0 / (HEADS * N)) + b2_ref[...]
        z = jnp.dot(pooled, wo1_ref[...], preferred_element_type=jnp.float32) + bo1_ref[...]
        z = jnp.maximum(z, 0.0)
        z = jnp.dot(z, wo2_ref[...], preferred_element_type=jnp.float32) + bo2_ref[...]
        z = jnp.maximum(z, 0.0)
        z = jnp.dot(z, wo3_ref[...], preferred_element_type=jnp.float32) + bo3_ref[...]
        z = z - jnp.max(z, axis=1, keepdims=True)
        ez = jnp.exp(z)
        out_ref[...] = ez / jnp.sum(ez, axis=1, keepdims=True)


def _tc_fin(oA, oB, ssA, ssB, bias2, Wo1, bo1, Wo2, bo2, Wo3, bo3):
    return pl.pallas_call(
        _kfin_body,
        grid=(NP // BN,),
        in_specs=[
            pl.BlockSpec((BN, 256), lambda i: (i, 0)),
            pl.BlockSpec((BN, 256), lambda i: (i, 0)),
            pl.BlockSpec((256, HID), lambda i: (0, 0)),
            pl.BlockSpec((256, HID), lambda i: (0, 0)),
            pl.BlockSpec((1, HID), lambda i: (0, 0)),
            pl.BlockSpec((HID, HID), lambda i: (0, 0)),
            pl.BlockSpec((1, HID), lambda i: (0, 0)),
            pl.BlockSpec((HID, HID // 2), lambda i: (0, 0)),
            pl.BlockSpec((1, HID // 2), lambda i: (0, 0)),
            pl.BlockSpec((HID // 2, 3), lambda i: (0, 0)),
            pl.BlockSpec((1, 3), lambda i: (0, 0)),
        ],
        out_specs=pl.BlockSpec((1, 3), lambda i: (0, 0)),
        out_shape=_SDS((1, 3), jnp.float32),
        scratch_shapes=[pltpu.VMEM((1, HID), jnp.float32)],
    )(oA, oB, ssA, ssB, bias2.reshape(1, HID), Wo1, bo1.reshape(1, HID),
      Wo2, bo2.reshape(1, HID // 2), Wo3, bo3.reshape(1, 3))


def _asad_mat(a_src, a_dst):
    eye = jnp.eye(HEADS, dtype=jnp.float32)
    s = (eye[:, None, :] * a_src[:, :, None]).reshape(HH, HEADS)
    d = (eye[:, None, :] * a_dst[:, :, None]).reshape(HH, HEADS)
    return jnp.concatenate([s, d], axis=1)


def kernel(x, edge_index, W_in, b_in, W0, as0, ad0, bias0, W1, as1, ad1,
           bias1, W2, as2, ad2, bias2, Wo1, bo1, Wo2, bo2, Wo3, bo3):
    x_p = jnp.pad(x, ((0, NP - N), (0, 0)))
    lsrc, ldl, cnts = _edge_scan(edge_index[0], edge_index[1])

    def edge_phase(hpA, hpB, sa):
        adstf = lax.optimization_barrier(sa[:, 8:16].reshape(-1))
        oA, oB = _gat_edge_phase(hpA, hpB, sa, adstf, lsrc, ldl, cnts)
        return lax.optimization_barrier(oA), lax.optimization_barrier(oB)

    p = jnp.arange(HH) % HEADS * HID + jnp.arange(HH) // HEADS
    hpA, hpB, sa = _tc_in(x_p, W_in, b_in, W0[:, p], _asad_mat(as0, ad0)[p])
    oA, oB = edge_phase(hpA, hpB, sa)
    hpA, hpB, sa = _tc_mid(oA, oB, bias0[p], W1[p][:, p], _asad_mat(as1, ad1)[p])
    oA, oB = edge_phase(hpA, hpB, sa)
    hpA, hpB, sa = _tc_mid(oA, oB, bias1[p], W2[p][:, p], _asad_mat(as2, ad2)[p])
    oA, oB = edge_phase(hpA, hpB, sa)

    ssum = jnp.repeat(jnp.eye(HID, dtype=jnp.float32), HEADS, axis=0)
    return _tc_fin(oA, oB, ssum[:256], ssum[256:], bias2, Wo1, bo1, Wo2, bo2,
                   Wo3, bo3)


# speedup vs baseline: 12.7699x; 1.0529x over previous
import functools

import jax
import jax.numpy as jnp
from jax import lax
from jax.experimental import pallas as pl
from jax.experimental.pallas import tpu as pltpu
from jax.experimental.pallas import tpu_sc as plsc

N = 10000
NP = 10016
F_IN = 128
HID = 64
HEADS = 8
HH = HID * HEADS
E = 160000
NT = 32
DPT = 313
CAP = 163840
FB = 2048
BN = 2504

_SC_PARAMS = pltpu.CompilerParams(needs_layout_passes=False)
_SDS = jax.ShapeDtypeStruct


def _wid():
    return lax.axis_index("s") * 2 + lax.axis_index("c")


def _scan_body(src_hbm, dst_hbm, lsrc_hbm, ldl_hbm, cnts_hbm,
               sbuf, dbuf, stg_s, stg_d, cntv):
    t = _wid()
    lo = t * DPT
    lanes = lax.iota(jnp.int32, 16)

    def append16(svec, dlvec, mask, pos_fb):
        pos, fbase = pos_fb
        mi = mask.astype(jnp.int32)
        ofs = plsc.cumsum(mi) - 1
        cnt = jnp.sum(mi, axis=0)
        plsc.store_scatter(stg_s, [pos + ofs], svec, mask=mask)
        plsc.store_scatter(stg_d, [pos + ofs], dlvec, mask=mask)
        pos = pos + cnt
        do_f = pos >= FB

        @pl.when(do_f)
        def _():
            base = pl.multiple_of(t * CAP + fbase, FB)
            pltpu.sync_copy(stg_s.at[pl.ds(0, FB)], lsrc_hbm.at[pl.ds(base, FB)])
            pltpu.sync_copy(stg_d.at[pl.ds(0, FB)], ldl_hbm.at[pl.ds(base, FB)])
            ts_ = stg_s[pl.ds(FB, 16)]
            td_ = stg_d[pl.ds(FB, 16)]
            stg_s[pl.ds(0, 16)] = ts_
            stg_d[pl.ds(0, 16)] = td_

        pos = jnp.where(do_f, pos - FB, pos)
        fbase = jnp.where(do_f, fbase + FB, fbase)
        return pos, fbase

    C = 2048

    def chunk_body(nloc, pos_fb):
        def grp(g, pf):
            s16 = sbuf[pl.ds(g * 16, 16)]
            d16 = dbuf[pl.ds(g * 16, 16)]
            dl = d16 - lo
            m = jnp.logical_and(dl >= 0, dl < DPT)
            return append16(s16, dl, m, pf)
        return lax.fori_loop(0, nloc // 16, grp, pos_fb)

    def full_chunk(c, pf):
        base = pl.multiple_of(c * C, C)
        pltpu.sync_copy(src_hbm.at[pl.ds(base, C)], sbuf)
        pltpu.sync_copy(dst_hbm.at[pl.ds(base, C)], dbuf)
        return chunk_body(C, pf)

    pf = lax.fori_loop(0, E // C, full_chunk, (jnp.int32(0), jnp.int32(0)))
    TAIL = E - (E // C) * C
    if TAIL:
        pltpu.sync_copy(src_hbm.at[pl.ds(E - TAIL, TAIL)], sbuf.at[pl.ds(0, TAIL)])
        pltpu.sync_copy(dst_hbm.at[pl.ds(E - TAIL, TAIL)], dbuf.at[pl.ds(0, TAIL)])
        pf = chunk_body(TAIL, pf)

    def selfloop(g, pf):
        dl = g * 16 + lanes
        dglob = lo + dl
        m = jnp.logical_and(dl < DPT, dglob < N)
        return append16(dglob, dl, m, pf)
    pf = lax.fori_loop(0, (DPT + 15) // 16, selfloop, pf)

    pos, fbase = pf
    total = pos + fbase
    target = jnp.bitwise_and(total + 127, jnp.int32(~127))
    k = target - total

    def padgrp(it, pf):
        m = (it * 16 + lanes) < k
        return append16(jnp.zeros((16,), jnp.int32),
                        jnp.full((16,), DPT, jnp.int32), m, pf)
    pos, fbase = lax.fori_loop(0, 8, padgrp, (pos, fbase))

    def fflush(kk, c):
        base = pl.multiple_of(t * CAP + fbase + kk * 128, 128)
        pltpu.sync_copy(stg_s.at[pl.ds(kk * 128, 128)], lsrc_hbm.at[pl.ds(base, 128)])
        pltpu.sync_copy(stg_d.at[pl.ds(kk * 128, 128)], ldl_hbm.at[pl.ds(base, 128)])
        return c
    lax.fori_loop(0, pos // 128, fflush, 0)

    cntv[...] = jnp.zeros((16,), jnp.int32) + (fbase + pos)
    pltpu.sync_copy(cntv, cnts_hbm.at[t])


def _edge_scan(src, dst):
    mesh = plsc.VectorSubcoreMesh(core_axis_name="c", subcore_axis_name="s")
    scan = pl.kernel(
        _scan_body,
        out_type=(
            _SDS((NT * CAP,), jnp.int32),
            _SDS((NT * CAP,), jnp.int32),
            _SDS((NT, 16), jnp.int32),
        ),
        mesh=mesh,
        compiler_params=_SC_PARAMS,
        scratch_types=[
            pltpu.VMEM((2048,), jnp.int32),
            pltpu.VMEM((2048,), jnp.int32),
            pltpu.VMEM((FB + 32,), jnp.int32),
            pltpu.VMEM((FB + 32,), jnp.int32),
            pltpu.VMEM((16,), jnp.int32),
        ],
    )
    return scan(src, dst)


def _layer_body(hpA_hbm, hpB_hbm, sa_hbm, adstf_hbm, lsrc_hbm, ldl_hbm,
                cnts_hbm, outA_hbm, outB_hbm, exb_hbm,
                adst_own, den, cntv, sidx, dlv, srows, exstage,
                gi0, gi1, rows0, rows1, exv, dlh, acc, sem0, sem1):
    t = _wid()
    lo = t * DPT
    lanes = lax.iota(jnp.int32, 16)
    lane7 = jnp.bitwise_and(lanes, 7)
    mlo = lanes < 8
    lbase = t * CAP

    pltpu.sync_copy(cnts_hbm.at[t], cntv)
    npad = cntv[...][0]

    adst_own[pl.ds(DPT * 8 - 8, 16)] = jnp.zeros((16,), jnp.float32)
    pltpu.sync_copy(adstf_hbm.at[pl.ds(pl.multiple_of(lo * 8, 8), DPT * 8)], adst_own.at[pl.ds(0, DPT * 8)])

    def dzero(i, c):
        den[pl.ds(i * 16, 16)] = jnp.zeros((16,), jnp.float32)
        return c
    lax.fori_loop(0, (DPT + 1) * 8 // 16, dzero, 0)

    def p1_chunk(c, carry):
        base = pl.multiple_of(c * 64, 64)
        lb = pl.multiple_of(lbase + base, 64)
        pltpu.sync_copy(lsrc_hbm.at[pl.ds(lb, 64)], sidx)
        pltpu.sync_copy(ldl_hbm.at[pl.ds(lb, 64)], dlv)
        pltpu.async_copy(sa_hbm.at[sidx], srows, sem0).wait()

        def edge1(g, c2):
            for j in range(4):
                e = g * 4 + j
                es = jnp.zeros((16,), jnp.int32) + e
                srow = srows[e, pl.ds(0, 16)]
                dsp = plsc.load_gather(dlv, [es])
                aidx = dsp * 8 + lane7
                adv = plsc.load_gather(adst_own, [aidx])
                s16 = srow + adv
                e16 = jnp.where(s16 > 0, s16, jnp.float32(0.2) * s16)
                ex = jnp.exp(e16)
                plsc.addupdate_scatter(den, [aidx], ex, mask=mlo)
                plsc.store_scatter(exstage, [es * 8 + lane7], ex, mask=mlo)
            return c2
        lax.fori_loop(0, 16, edge1, 0)
        pltpu.sync_copy(exstage, exb_hbm.at[pl.ds(pl.multiple_of((lbase + base) * 8, 512), 512)])
        return carry

    lax.fori_loop(0, npad // 64, p1_chunk, 0)

    def p2(i, c):
        v = den[pl.ds(i * 16, 16)]
        den[pl.ds(i * 16, 16)] = jnp.float32(1.0) / (v + jnp.float32(1e-16))
        return c
    lax.fori_loop(0, (DPT + 1) * 8 // 16, p2, 0)

    for r, (hp_hbm, out_hbm) in enumerate(((hpA_hbm, outA_hbm), (hpB_hbm, outB_hbm))):
        def azero(i, c):
            for u in range(4):
                acc[pl.ds(i * 64 + u * 16, 16)] = jnp.zeros((16,), jnp.float32)
            return c
        lax.fori_loop(0, (DPT + 1) * 256 // 64, azero, 0)

        pltpu.sync_copy(lsrc_hbm.at[pl.ds(pl.multiple_of(lbase, 64), 64)], gi0)
        pltpu.make_async_copy(hp_hbm.at[gi0], rows0, sem0).start()

        def process_half(eb, rows):
            eb2 = pl.multiple_of(lbase + eb, 64)
            pltpu.sync_copy(ldl_hbm.at[pl.ds(eb2, 64)], dlh)
            pltpu.sync_copy(exb_hbm.at[pl.ds(pl.multiple_of((lbase + eb) * 8, 512), 512)], exv)

            def edge3(g, c2):
                for j in range(4):
                    e = g * 4 + j
                    es = jnp.zeros((16,), jnp.int32) + e
                    dsp = plsc.load_gather(dlh, [es])
                    exe = plsc.load_gather(exv, [es * 8 + lane7])
                    rdv = plsc.load_gather(den, [dsp * 8 + lane7])
                    alpha = exe * rdv
                    abase = dsp * 256
                    for v in range(16):
                        col = lanes + v * 16
                        rv = rows[e, pl.ds(v * 16, 16)]
                        plsc.addupdate_scatter(acc, [abase + col], alpha * rv)
                return c2
            lax.fori_loop(0, 16, edge3, 0)

        def p3_iter(i, carry):
            base = pl.multiple_of(i * 128, 128)
            pltpu.sync_copy(lsrc_hbm.at[pl.ds(pl.multiple_of(lbase + base + 64, 64), 64)], gi1)
            pltpu.make_async_copy(hp_hbm.at[gi1], rows1, sem1).start()
            pltpu.make_async_copy(hp_hbm.at[gi0], rows0, sem0).wait()
            process_half(base, rows0)
            @pl.when(base + 128 < npad)
            def _():
                pltpu.sync_copy(lsrc_hbm.at[pl.ds(pl.multiple_of(lbase + base + 128, 64), 64)], gi0)
                pltpu.make_async_copy(hp_hbm.at[gi0], rows0, sem0).start()
            pltpu.make_async_copy(hp_hbm.at[gi1], rows1, sem1).wait()
            process_half(base + 64, rows1)
            return carry

        lax.fori_loop(0, npad // 128, p3_iter, 0)
        pltpu.sync_copy(acc.at[pl.ds(0, DPT * 256)],
                        out_hbm.at[pl.ds(pl.multiple_of(lo * 256, 128), DPT * 256)])


def _gat_edge_phase(hpA, hpB, sa, adstf, lsrc, ldl, cnts):
    mesh = plsc.VectorSubcoreMesh(core_axis_name="c", subcore_axis_name="s")
    layer = pl.kernel(
        _layer_body,
        out_type=(
            _SDS((NP * 256,), jnp.float32),
            _SDS((NP * 256,), jnp.float32),
            _SDS((NT * CAP * 8,), jnp.float32),
        ),
        mesh=mesh,
        compiler_params=_SC_PARAMS,
        scratch_types=[
            pltpu.VMEM(((DPT + 1) * 8,), jnp.float32),
            pltpu.VMEM(((DPT + 1) * 8,), jnp.float32),
            pltpu.VMEM((16,), jnp.int32),
            pltpu.VMEM((64,), jnp.int32),
            pltpu.VMEM((64,), jnp.int32),
            pltpu.VMEM((64, 128), jnp.float32),
            pltpu.VMEM((512,), jnp.float32),
            pltpu.VMEM((64,), jnp.int32),
            pltpu.VMEM((64,), jnp.int32),
            pltpu.VMEM((64, 256), jnp.float32),
            pltpu.VMEM((64, 256), jnp.float32),
            pltpu.VMEM((512,), jnp.float32),
            pltpu.VMEM((64,), jnp.int32),
            pltpu.VMEM(((DPT + 1) * 256,), jnp.float32),
            pltpu.SemaphoreType.DMA,
            pltpu.SemaphoreType.DMA,
        ],
    )
    outA, outB, _ = layer(hpA, hpB, sa, adstf, lsrc, ldl, cnts)
    return outA.reshape(NP, 256), outB.reshape(NP, 256)


def _kin_body(x_ref, win_ref, bin_ref, w0_ref, asad_ref, hpA_ref, hpB_ref, sa_ref):
    h = jnp.dot(x_ref[...], win_ref[...], preferred_element_type=jnp.float32)
    h = h + bin_ref[...]
    hp = jnp.dot(h, w0_ref[...], preferred_element_type=jnp.float32)
    sa = jnp.dot(hp, asad_ref[...], preferred_element_type=jnp.float32,
                 precision=lax.Precision.HIGHEST)
    hpA_ref[...] = hp[:, :256]
    hpB_ref[...] = hp[:, 256:]
    sa_ref[...] = jnp.concatenate(
        [sa, jnp.zeros((sa.shape[0], 112), jnp.float32)], axis=1)


def _tc_in(x_p, W_in, b_in, W0, asad):
    return pl.pallas_call(
        _kin_body,
        grid=(NP // BN,),
        in_specs=[
            pl.BlockSpec((BN, F_IN), lambda i: (i, 0)),
            pl.BlockSpec((F_IN, HID), lambda i: (0, 0)),
            pl.BlockSpec((1, HID), lambda i: (0, 0)),
            pl.BlockSpec((HID, HH), lambda i: (0, 0)),
            pl.BlockSpec((HH, 16), lambda i: (0, 0)),
        ],
        out_specs=[
            pl.BlockSpec((BN, 256), lambda i: (i, 0)),
            pl.BlockSpec((BN, 256), lambda i: (i, 0)),
            pl.BlockSpec((BN, 128), lambda i: (i, 0)),
        ],
        out_shape=[
            _SDS((NP, 256), jnp.float32),
            _SDS((NP, 256), jnp.float32),
            _SDS((NP, 128), jnp.float32),
        ],
    )(x_p, W_in, b_in.reshape(1, HID), W0, asad)


def _kmid_body(pA_ref, pB_ref, bA_ref, bB_ref, w_ref, asad_ref,
               hpA_ref, hpB_ref, sa_ref):
    vA = pA_ref[...] + bA_ref[...]
    vB = pB_ref[...] + bB_ref[...]
    aA = jnp.where(vA > 0, vA, jnp.exp(jnp.minimum(vA, 0.0)) - 1.0)
    aB = jnp.where(vB > 0, vB, jnp.exp(jnp.minimum(vB, 0.0)) - 1.0)
    act = jnp.concatenate([aA, aB], axis=1)
    hp = jnp.dot(act, w_ref[...], preferred_element_type=jnp.float32)
    sa = jnp.dot(hp, asad_ref[...], preferred_element_type=jnp.float32,
                 precision=lax.Precision.HIGHEST)
    hpA_ref[...] = hp[:, :256]
    hpB_ref[...] = hp[:, 256:]
    sa_ref[...] = jnp.concatenate(
        [sa, jnp.zeros((sa.shape[0], 112), jnp.float32)], axis=1)


def _tc_mid(prevA, prevB, bias, W, asad):
    return pl.pallas_call(
        _kmid_body,
        grid=(NP // BN,),
        in_specs=[
            pl.BlockSpec((BN, 256), lambda i: (i, 0)),
            pl.BlockSpec((BN, 256), lambda i: (i, 0)),
            pl.BlockSpec((1, 256), lambda i: (0, 0)),
            pl.BlockSpec((1, 256), lambda i: (0, 0)),
            pl.BlockSpec((HH, HH), lambda i: (0, 0)),
            pl.BlockSpec((HH, 16), lambda i: (0, 0)),
        ],
        out_specs=[
            pl.BlockSpec((BN, 256), lambda i: (i, 0)),
            pl.BlockSpec((BN, 256), lambda i: (i, 0)),
            pl.BlockSpec((BN, 128), lambda i: (i, 0)),
        ],
        out_shape=[
            _SDS((NP, 256), jnp.float32),
            _SDS((NP, 256), jnp.float32),
            _SDS((NP, 128), jnp.float32),
        ],
    )(prevA, prevB, bias[:256].reshape(1, 256), bias[256:].reshape(1, 256),
      W, asad)


def _kfin_body(oA_ref, oB_ref, ssA_ref, ssB_ref, b2_ref, wo1_ref, bo1_ref,
               wo2_ref, bo2_ref, wo3_ref, bo3_ref, out_ref, acc):
    i = pl.program_id(0)
    s = (jnp.dot(oA_ref[...], ssA_ref[...], preferred_element_type=jnp.float32,
                 precision=lax.Precision.HIGHEST)
         + jnp.dot(oB_ref[...], ssB_ref[...], preferred_element_type=jnp.float32,
                   precision=lax.Precision.HIGHEST))
    gid = lax.broadcasted_iota(jnp.int32, (BN, 1), 0) + i * BN
    s = jnp.where(gid < N, s, jnp.float32(0.0))
    part = jnp.sum(s, axis=0, keepdims=True)

    @pl.when(i == 0)
    def _():
        acc[...] = part

    @pl.when(i > 0)
    def _():
        acc[...] = acc[...] + part

    @pl.when(i == NP // BN - 1)
    def _():
        pooled = acc[...] * jnp.float32(1.0 / (HEADS * N)) + b2_ref[...]
        z = jnp.dot(pooled, wo1_ref[...], preferred_element_type=jnp.float32) + bo1_ref[...]
        z = jnp.maximum(z, 0.0)
        z = jnp.dot(z, wo2_ref[...], preferred_element_type=jnp.float32) + bo2_ref[...]
        z = jnp.maximum(z, 0.0)
        z = jnp.dot(z, wo3_ref[...], preferred_element_type=jnp.float32) + bo3_ref[...]
        z = z - jnp.max(z, axis=1, keepdims=True)
        ez = jnp.exp(z)
        out_ref[...] = ez / jnp.sum(ez, axis=1, keepdims=True)


def _tc_fin(oA, oB, ssA, ssB, bias2, Wo1, bo1, Wo2, bo2, Wo3, bo3):
    return pl.pallas_call(
        _kfin_body,
        grid=(NP // BN,),
        in_specs=[
            pl.BlockSpec((BN, 256), lambda i: (i, 0)),
            pl.BlockSpec((BN, 256), lambda i: (i, 0)),
            pl.BlockSpec((256, HID), lambda i: (0, 0)),
            pl.BlockSpec((256, HID), lambda i: (0, 0)),
            pl.BlockSpec((1, HID), lambda i: (0, 0)),
            pl.BlockSpec((HID, HID), lambda i: (0, 0)),
            pl.BlockSpec((1, HID), lambda i: (0, 0)),
            pl.BlockSpec((HID, HID // 2), lambda i: (0, 0)),
            pl.BlockSpec((1, HID // 2), lambda i: (0, 0)),
            pl.BlockSpec((HID // 2, 3), lambda i: (0, 0)),
            pl.BlockSpec((1, 3), lambda i: (0, 0)),
        ],
        out_specs=pl.BlockSpec((1, 3), lambda i: (0, 0)),
        out_shape=_SDS((1, 3), jnp.float32),
        scratch_shapes=[pltpu.VMEM((1, HID), jnp.float32)],
    )(oA, oB, ssA, ssB, bias2.reshape(1, HID), Wo1, bo1.reshape(1, HID),
      Wo2, bo2.reshape(1, HID // 2), Wo3, bo3.reshape(1, 3))


def _asad_mat(a_src, a_dst):
    eye = jnp.eye(HEADS, dtype=jnp.float32)
    s = (eye[:, None, :] * a_src[:, :, None]).reshape(HH, HEADS)
    d = (eye[:, None, :] * a_dst[:, :, None]).reshape(HH, HEADS)
    return jnp.concatenate([s, d], axis=1)


def kernel(x, edge_index, W_in, b_in, W0, as0, ad0, bias0, W1, as1, ad1,
           bias1, W2, as2, ad2, bias2, Wo1, bo1, Wo2, bo2, Wo3, bo3):
    x_p = jnp.pad(x, ((0, NP - N), (0, 0)))
    lsrc, ldl, cnts = _edge_scan(edge_index[0], edge_index[1])

    def edge_phase(hpA, hpB, sa):
        adstf = lax.optimization_barrier(sa[:, 8:16].reshape(-1))
        oA, oB = _gat_edge_phase(hpA, hpB, sa, adstf, lsrc, ldl, cnts)
        return lax.optimization_barrier(oA), lax.optimization_barrier(oB)

    p = jnp.arange(HH) % HEADS * HID + jnp.arange(HH) // HEADS
    hpA, hpB, sa = _tc_in(x_p, W_in, b_in, W0[:, p], _asad_mat(as0, ad0)[p])
    oA, oB = edge_phase(hpA, hpB, sa)
    hpA, hpB, sa = _tc_mid(oA, oB, bias0[p], W1[p][:, p], _asad_mat(as1, ad1)[p])
    oA, oB = edge_phase(hpA, hpB, sa)
    hpA, hpB, sa = _tc_mid(oA, oB, bias1[p], W2[p][:, p], _asad_mat(as2, ad2)[p])
    oA, oB = edge_phase(hpA, hpB, sa)

    ssum = jnp.repeat(jnp.eye(HID, dtype=jnp.float32), HEADS, axis=0)
    return _tc_fin(oA, oB, ssum[:256], ssum[256:], bias2, Wo1, bo1, Wo2, bo2,
                   Wo3, bo3)


# speedup vs baseline: 13.0575x; 1.0225x over previous
import functools

import jax
import jax.numpy as jnp
from jax import lax
from jax.experimental import pallas as pl
from jax.experimental.pallas import tpu as pltpu
from jax.experimental.pallas import tpu_sc as plsc

N = 10000
NP = 10016
F_IN = 128
HID = 64
HEADS = 8
HH = HID * HEADS
E = 160000
NT = 32
DPT = 313
CAP = 163840
FB = 2048
BN = 2504

_SC_PARAMS = pltpu.CompilerParams(needs_layout_passes=False)
_SDS = jax.ShapeDtypeStruct


def _wid():
    return lax.axis_index("s") * 2 + lax.axis_index("c")


def _scan_body(src_hbm, dst_hbm, lsrc_hbm, ldl_hbm, cnts_hbm,
               sbuf, dbuf, sbuf2, dbuf2, stg_s, stg_d, cntv, ssem0, ssem1):
    t = _wid()
    lo = t * DPT
    lanes = lax.iota(jnp.int32, 16)

    def append16(svec, dlvec, mask, pos_fb):
        pos, fbase = pos_fb
        mi = mask.astype(jnp.int32)
        ofs = plsc.cumsum(mi) - 1
        cnt = jnp.sum(mi, axis=0)
        plsc.store_scatter(stg_s, [pos + ofs], svec, mask=mask)
        plsc.store_scatter(stg_d, [pos + ofs], dlvec, mask=mask)
        pos = pos + cnt
        do_f = pos >= FB

        @pl.when(do_f)
        def _():
            base = pl.multiple_of(t * CAP + fbase, FB)
            pltpu.sync_copy(stg_s.at[pl.ds(0, FB)], lsrc_hbm.at[pl.ds(base, FB)])
            pltpu.sync_copy(stg_d.at[pl.ds(0, FB)], ldl_hbm.at[pl.ds(base, FB)])
            ts_ = stg_s[pl.ds(FB, 16)]
            td_ = stg_d[pl.ds(FB, 16)]
            stg_s[pl.ds(0, 16)] = ts_
            stg_d[pl.ds(0, 16)] = td_

        pos = jnp.where(do_f, pos - FB, pos)
        fbase = jnp.where(do_f, fbase + FB, fbase)
        return pos, fbase

    C = 4096

    def chunk_body(nloc, pos_fb, sb, db):
        def grp(g, pf):
            s16 = sb[pl.ds(pl.multiple_of(g * 16, 16), 16)]
            d16 = db[pl.ds(pl.multiple_of(g * 16, 16), 16)]
            dl = d16 - lo
            m = jnp.logical_and(dl >= 0, dl < DPT)
            return append16(s16, dl, m, pf)
        return lax.fori_loop(0, nloc // 16, grp, pos_fb)

    NFULL = E // C
    pltpu.make_async_copy(src_hbm.at[pl.ds(0, C)], sbuf, ssem0).start()
    pltpu.make_async_copy(dst_hbm.at[pl.ds(0, C)], dbuf, ssem0).start()

    def full_pair(i, pf):
        baseA = pl.multiple_of(i * 2 * C, C)
        baseB = pl.multiple_of(i * 2 * C + C, C)
        pltpu.make_async_copy(src_hbm.at[pl.ds(baseB, C)], sbuf2, ssem1).start()
        pltpu.make_async_copy(dst_hbm.at[pl.ds(baseB, C)], dbuf2, ssem1).start()
        pltpu.make_async_copy(src_hbm.at[pl.ds(baseA, C)], sbuf, ssem0).wait()
        pltpu.make_async_copy(dst_hbm.at[pl.ds(baseA, C)], dbuf, ssem0).wait()
        pf = chunk_body(C, pf, sbuf, dbuf)

        @pl.when(i + 1 < NFULL // 2)
        def _():
            baseC = pl.multiple_of(i * 2 * C + 2 * C, C)
            pltpu.make_async_copy(src_hbm.at[pl.ds(baseC, C)], sbuf, ssem0).start()
            pltpu.make_async_copy(dst_hbm.at[pl.ds(baseC, C)], dbuf, ssem0).start()
        pltpu.make_async_copy(src_hbm.at[pl.ds(baseB, C)], sbuf2, ssem1).wait()
        pltpu.make_async_copy(dst_hbm.at[pl.ds(baseB, C)], dbuf2, ssem1).wait()
        return chunk_body(C, pf, sbuf2, dbuf2)

    NPAIR = NFULL // 2
    pf = lax.fori_loop(0, NPAIR, full_pair, (jnp.int32(0), jnp.int32(0)))
    done = NPAIR * 2 * C
    while done < E:
        step = min(C, E - done)
        pltpu.sync_copy(src_hbm.at[pl.ds(done, step)], sbuf.at[pl.ds(0, step)])
        pltpu.sync_copy(dst_hbm.at[pl.ds(done, step)], dbuf.at[pl.ds(0, step)])
        pf = chunk_body(step, pf, sbuf, dbuf)
        done += step

    def selfloop(g, pf):
        dl = g * 16 + lanes
        dglob = lo + dl
        m = jnp.logical_and(dl < DPT, dglob < N)
        return append16(dglob, dl, m, pf)
    pf = lax.fori_loop(0, (DPT + 15) // 16, selfloop, pf)

    pos, fbase = pf
    total = pos + fbase
    target = jnp.bitwise_and(total + 127, jnp.int32(~127))
    k = target - total

    def padgrp(it, pf):
        m = (it * 16 + lanes) < k
        return append16(jnp.zeros((16,), jnp.int32),
                        jnp.full((16,), DPT, jnp.int32), m, pf)
    pos, fbase = lax.fori_loop(0, 8, padgrp, (pos, fbase))

    def fflush(kk, c):
        base = pl.multiple_of(t * CAP + fbase + kk * 128, 128)
        pltpu.sync_copy(stg_s.at[pl.ds(kk * 128, 128)], lsrc_hbm.at[pl.ds(base, 128)])
        pltpu.sync_copy(stg_d.at[pl.ds(kk * 128, 128)], ldl_hbm.at[pl.ds(base, 128)])
        return c
    lax.fori_loop(0, pos // 128, fflush, 0)

    cntv[...] = jnp.zeros((16,), jnp.int32) + (fbase + pos)
    pltpu.sync_copy(cntv, cnts_hbm.at[t])


def _edge_scan(src, dst):
    mesh = plsc.VectorSubcoreMesh(core_axis_name="c", subcore_axis_name="s")
    scan = pl.kernel(
        _scan_body,
        out_type=(
            _SDS((NT * CAP,), jnp.int32),
            _SDS((NT * CAP,), jnp.int32),
            _SDS((NT, 16), jnp.int32),
        ),
        mesh=mesh,
        compiler_params=_SC_PARAMS,
        scratch_types=[
            pltpu.VMEM((4096,), jnp.int32),
            pltpu.VMEM((4096,), jnp.int32),
            pltpu.VMEM((4096,), jnp.int32),
            pltpu.VMEM((4096,), jnp.int32),
            pltpu.VMEM((FB + 32,), jnp.int32),
            pltpu.VMEM((FB + 32,), jnp.int32),
            pltpu.VMEM((16,), jnp.int32),
            pltpu.SemaphoreType.DMA,
            pltpu.SemaphoreType.DMA,
        ],
    )
    return scan(src, dst)


def _layer_body(hpA_hbm, hpB_hbm, sa_hbm, adstf_hbm, lsrc_hbm, ldl_hbm,
                cnts_hbm, outA_hbm, outB_hbm, exb_hbm,
                adst_own, den, cntv, sidx, dlv, srows, exstage,
                gi0, gi1, rows0, rows1, exv, dlh, acc, sem0, sem1):
    t = _wid()
    lo = t * DPT
    lanes = lax.iota(jnp.int32, 16)
    lane7 = jnp.bitwise_and(lanes, 7)
    mlo = lanes < 8
    lbase = t * CAP

    pltpu.sync_copy(cnts_hbm.at[t], cntv)
    npad = cntv[...][0]

    adst_own[pl.ds(DPT * 8 - 8, 16)] = jnp.zeros((16,), jnp.float32)
    pltpu.sync_copy(adstf_hbm.at[pl.ds(pl.multiple_of(lo * 8, 8), DPT * 8)], adst_own.at[pl.ds(0, DPT * 8)])

    def dzero(i, c):
        den[pl.ds(i * 16, 16)] = jnp.zeros((16,), jnp.float32)
        return c
    lax.fori_loop(0, (DPT + 1) * 8 // 16, dzero, 0)

    def p1_chunk(c, carry):
        base = pl.multiple_of(c * 64, 64)
        lb = pl.multiple_of(lbase + base, 64)
        pltpu.sync_copy(lsrc_hbm.at[pl.ds(lb, 64)], sidx)
        pltpu.sync_copy(ldl_hbm.at[pl.ds(lb, 64)], dlv)
        pltpu.async_copy(sa_hbm.at[sidx], srows, sem0).wait()

        def edge1(g, c2):
            for j in range(4):
                e = g * 4 + j
                es = jnp.zeros((16,), jnp.int32) + e
                srow = srows[e, pl.ds(0, 16)]
                dsp = plsc.load_gather(dlv, [es])
                aidx = dsp * 8 + lane7
                adv = plsc.load_gather(adst_own, [aidx])
                s16 = srow + adv
                e16 = jnp.where(s16 > 0, s16, jnp.float32(0.2) * s16)
                ex = jnp.exp(e16)
                plsc.addupdate_scatter(den, [aidx], ex, mask=mlo)
                plsc.store_scatter(exstage, [es * 8 + lane7], ex, mask=mlo)
            return c2
        lax.fori_loop(0, 16, edge1, 0)
        pltpu.sync_copy(exstage, exb_hbm.at[pl.ds(pl.multiple_of((lbase + base) * 8, 512), 512)])
        return carry

    lax.fori_loop(0, npad // 64, p1_chunk, 0)

    def p2(i, c):
        v = den[pl.ds(i * 16, 16)]
        den[pl.ds(i * 16, 16)] = jnp.float32(1.0) / (v + jnp.float32(1e-16))
        return c
    lax.fori_loop(0, (DPT + 1) * 8 // 16, p2, 0)

    for r, (hp_hbm, out_hbm) in enumerate(((hpA_hbm, outA_hbm), (hpB_hbm, outB_hbm))):
        def azero(i, c):
            for u in range(4):
                acc[pl.ds(i * 64 + u * 16, 16)] = jnp.zeros((16,), jnp.float32)
            return c
        lax.fori_loop(0, (DPT + 1) * 256 // 64, azero, 0)

        pltpu.sync_copy(lsrc_hbm.at[pl.ds(pl.multiple_of(lbase, 64), 64)], gi0)
        pltpu.make_async_copy(hp_hbm.at[gi0], rows0, sem0).start()

        def process_half(eb, rows):
            eb2 = pl.multiple_of(lbase + eb, 64)
            pltpu.sync_copy(ldl_hbm.at[pl.ds(eb2, 64)], dlh)
            pltpu.sync_copy(exb_hbm.at[pl.ds(pl.multiple_of((lbase + eb) * 8, 512), 512)], exv)

            def edge3(g, c2):
                for j in range(4):
                    e = g * 4 + j
                    es = jnp.zeros((16,), jnp.int32) + e
                    dsp = plsc.load_gather(dlh, [es])
                    exe = plsc.load_gather(exv, [es * 8 + lane7])
                    rdv = plsc.load_gather(den, [dsp * 8 + lane7])
                    alpha = exe * rdv
                    abase = dsp * 256
                    for v in range(16):
                        col = lanes + v * 16
                        rv = rows[e, pl.ds(v * 16, 16)]
                        plsc.addupdate_scatter(acc, [abase + col], alpha * rv)
                return c2
            lax.fori_loop(0, 16, edge3, 0)

        def p3_iter(i, carry):
            base = pl.multiple_of(i * 128, 128)
            pltpu.sync_copy(lsrc_hbm.at[pl.ds(pl.multiple_of(lbase + base + 64, 64), 64)], gi1)
            pltpu.make_async_copy(hp_hbm.at[gi1], rows1, sem1).start()
            pltpu.make_async_copy(hp_hbm.at[gi0], rows0, sem0).wait()
            process_half(base, rows0)
            @pl.when(base + 128 < npad)
            def _():
                pltpu.sync_copy(lsrc_hbm.at[pl.ds(pl.multiple_of(lbase + base + 128, 64), 64)], gi0)
                pltpu.make_async_copy(hp_hbm.at[gi0], rows0, sem0).start()
            pltpu.make_async_copy(hp_hbm.at[gi1], rows1, sem1).wait()
            process_half(base + 64, rows1)
            return carry

        lax.fori_loop(0, npad // 128, p3_iter, 0)
        pltpu.sync_copy(acc.at[pl.ds(0, DPT * 256)],
                        out_hbm.at[pl.ds(pl.multiple_of(lo * 256, 128), DPT * 256)])


def _gat_edge_phase(hpA, hpB, sa, adstf, lsrc, ldl, cnts):
    mesh = plsc.VectorSubcoreMesh(core_axis_name="c", subcore_axis_name="s")
    layer = pl.kernel(
        _layer_body,
        out_type=(
            _SDS((NP * 256,), jnp.float32),
            _SDS((NP * 256,), jnp.float32),
            _SDS((NT * CAP * 8,), jnp.float32),
        ),
        mesh=mesh,
        compiler_params=_SC_PARAMS,
        scratch_types=[
            pltpu.VMEM(((DPT + 1) * 8,), jnp.float32),
            pltpu.VMEM(((DPT + 1) * 8,), jnp.float32),
            pltpu.VMEM((16,), jnp.int32),
            pltpu.VMEM((64,), jnp.int32),
            pltpu.VMEM((64,), jnp.int32),
            pltpu.VMEM((64, 128), jnp.float32),
            pltpu.VMEM((512,), jnp.float32),
            pltpu.VMEM((64,), jnp.int32),
            pltpu.VMEM((64,), jnp.int32),
            pltpu.VMEM((64, 256), jnp.float32),
            pltpu.VMEM((64, 256), jnp.float32),
            pltpu.VMEM((512,), jnp.float32),
            pltpu.VMEM((64,), jnp.int32),
            pltpu.VMEM(((DPT + 1) * 256,), jnp.float32),
            pltpu.SemaphoreType.DMA,
            pltpu.SemaphoreType.DMA,
        ],
    )
    outA, outB, _ = layer(hpA, hpB, sa, adstf, lsrc, ldl, cnts)
    return outA.reshape(NP, 256), outB.reshape(NP, 256)


def _kin_body(x_ref, win_ref, bin_ref, w0_ref, asad_ref, hpA_ref, hpB_ref, sa_ref):
    h = jnp.dot(x_ref[...], win_ref[...], preferred_element_type=jnp.float32)
    h = h + bin_ref[...]
    hp = jnp.dot(h, w0_ref[...], preferred_element_type=jnp.float32)
    sa = jnp.dot(hp, asad_ref[...], preferred_element_type=jnp.float32,
                 precision=lax.Precision.HIGHEST)
    hpA_ref[...] = hp[:, :256]
    hpB_ref[...] = hp[:, 256:]
    sa_ref[...] = jnp.concatenate(
        [sa, jnp.zeros((sa.shape[0], 112), jnp.float32)], axis=1)


def _tc_in(x_p, W_in, b_in, W0, asad):
    return pl.pallas_call(
        _kin_body,
        grid=(NP // BN,),
        in_specs=[
            pl.BlockSpec((BN, F_IN), lambda i: (i, 0)),
            pl.BlockSpec((F_IN, HID), lambda i: (0, 0)),
            pl.BlockSpec((1, HID), lambda i: (0, 0)),
            pl.BlockSpec((HID, HH), lambda i: (0, 0)),
            pl.BlockSpec((HH, 16), lambda i: (0, 0)),
        ],
        out_specs=[
            pl.BlockSpec((BN, 256), lambda i: (i, 0)),
            pl.BlockSpec((BN, 256), lambda i: (i, 0)),
            pl.BlockSpec((BN, 128), lambda i: (i, 0)),
        ],
        out_shape=[
            _SDS((NP, 256), jnp.float32),
            _SDS((NP, 256), jnp.float32),
            _SDS((NP, 128), jnp.float32),
        ],
    )(x_p, W_in, b_in.reshape(1, HID), W0, asad)


def _kmid_body(pA_ref, pB_ref, bA_ref, bB_ref, w_ref, asad_ref,
               hpA_ref, hpB_ref, sa_ref):
    vA = pA_ref[...] + bA_ref[...]
    vB = pB_ref[...] + bB_ref[...]
    aA = jnp.where(vA > 0, vA, jnp.exp(jnp.minimum(vA, 0.0)) - 1.0)
    aB = jnp.where(vB > 0, vB, jnp.exp(jnp.minimum(vB, 0.0)) - 1.0)
    act = jnp.concatenate([aA, aB], axis=1)
    hp = jnp.dot(act, w_ref[...], preferred_element_type=jnp.float32)
    sa = jnp.dot(hp, asad_ref[...], preferred_element_type=jnp.float32,
                 precision=lax.Precision.HIGHEST)
    hpA_ref[...] = hp[:, :256]
    hpB_ref[...] = hp[:, 256:]
    sa_ref[...] = jnp.concatenate(
        [sa, jnp.zeros((sa.shape[0], 112), jnp.float32)], axis=1)


def _tc_mid(prevA, prevB, bias, W, asad):
    return pl.pallas_call(
        _kmid_body,
        grid=(NP // BN,),
        in_specs=[
            pl.BlockSpec((BN, 256), lambda i: (i, 0)),
            pl.BlockSpec((BN, 256), lambda i: (i, 0)),
            pl.BlockSpec((1, 256), lambda i: (0, 0)),
            pl.BlockSpec((1, 256), lambda i: (0, 0)),
            pl.BlockSpec((HH, HH), lambda i: (0, 0)),
            pl.BlockSpec((HH, 16), lambda i: (0, 0)),
        ],
        out_specs=[
            pl.BlockSpec((BN, 256), lambda i: (i, 0)),
            pl.BlockSpec((BN, 256), lambda i: (i, 0)),
            pl.BlockSpec((BN, 128), lambda i: (i, 0)),
        ],
        out_shape=[
            _SDS((NP, 256), jnp.float32),
            _SDS((NP, 256), jnp.float32),
            _SDS((NP, 128), jnp.float32),
        ],
    )(prevA, prevB, bias[:256].reshape(1, 256), bias[256:].reshape(1, 256),
      W, asad)


def _kfin_body(oA_ref, oB_ref, ssA_ref, ssB_ref, b2_ref, wo1_ref, bo1_ref,
               wo2_ref, bo2_ref, wo3_ref, bo3_ref, out_ref, acc):
    i = pl.program_id(0)
    s = (jnp.dot(oA_ref[...], ssA_ref[...], preferred_element_type=jnp.float32,
                 precision=lax.Precision.HIGHEST)
         + jnp.dot(oB_ref[...], ssB_ref[...], preferred_element_type=jnp.float32,
                   precision=lax.Precision.HIGHEST))
    gid = lax.broadcasted_iota(jnp.int32, (BN, 1), 0) + i * BN
    s = jnp.where(gid < N, s, jnp.float32(0.0))
    part = jnp.sum(s, axis=0, keepdims=True)

    @pl.when(i == 0)
    def _():
        acc[...] = part

    @pl.when(i > 0)
    def _():
        acc[...] = acc[...] + part

    @pl.when(i == NP // BN - 1)
    def _():
        pooled = acc[...] * jnp.float32(1.0 / (HEADS * N)) + b2_ref[...]
        z = jnp.dot(pooled, wo1_ref[...], preferred_element_type=jnp.float32) + bo1_ref[...]
        z = jnp.maximum(z, 0.0)
        z = jnp.dot(z, wo2_ref[...], preferred_element_type=jnp.float32) + bo2_ref[...]
        z = jnp.maximum(z, 0.0)
        z = jnp.dot(z, wo3_ref[...], preferred_element_type=jnp.float32) + bo3_ref[...]
        z = z - jnp.max(z, axis=1, keepdims=True)
        ez = jnp.exp(z)
        out_ref[...] = ez / jnp.sum(ez, axis=1, keepdims=True)


def _tc_fin(oA, oB, ssA, ssB, bias2, Wo1, bo1, Wo2, bo2, Wo3, bo3):
    return pl.pallas_call(
        _kfin_body,
        grid=(NP // BN,),
        in_specs=[
            pl.BlockSpec((BN, 256), lambda i: (i, 0)),
            pl.BlockSpec((BN, 256), lambda i: (i, 0)),
            pl.BlockSpec((256, HID), lambda i: (0, 0)),
            pl.BlockSpec((256, HID), lambda i: (0, 0)),
            pl.BlockSpec((1, HID), lambda i: (0, 0)),
            pl.BlockSpec((HID, HID), lambda i: (0, 0)),
            pl.BlockSpec((1, HID), lambda i: (0, 0)),
            pl.BlockSpec((HID, HID // 2), lambda i: (0, 0)),
            pl.BlockSpec((1, HID // 2), lambda i: (0, 0)),
            pl.BlockSpec((HID // 2, 3), lambda i: (0, 0)),
            pl.BlockSpec((1, 3), lambda i: (0, 0)),
        ],
        out_specs=pl.BlockSpec((1, 3), lambda i: (0, 0)),
        out_shape=_SDS((1, 3), jnp.float32),
        scratch_shapes=[pltpu.VMEM((1, HID), jnp.float32)],
    )(oA, oB, ssA, ssB, bias2.reshape(1, HID), Wo1, bo1.reshape(1, HID),
      Wo2, bo2.reshape(1, HID // 2), Wo3, bo3.reshape(1, 3))


def _asad_mat(a_src, a_dst):
    eye = jnp.eye(HEADS, dtype=jnp.float32)
    s = (eye[:, None, :] * a_src[:, :, None]).reshape(HH, HEADS)
    d = (eye[:, None, :] * a_dst[:, :, None]).reshape(HH, HEADS)
    return jnp.concatenate([s, d], axis=1)


def kernel(x, edge_index, W_in, b_in, W0, as0, ad0, bias0, W1, as1, ad1,
           bias1, W2, as2, ad2, bias2, Wo1, bo1, Wo2, bo2, Wo3, bo3):
    x_p = jnp.pad(x, ((0, NP - N), (0, 0)))
    lsrc, ldl, cnts = _edge_scan(edge_index[0], edge_index[1])

    def edge_phase(hpA, hpB, sa):
        adstf = lax.optimization_barrier(sa[:, 8:16].reshape(-1))
        oA, oB = _gat_edge_phase(hpA, hpB, sa, adstf, lsrc, ldl, cnts)
        return lax.optimization_barrier(oA), lax.optimization_barrier(oB)

    p = jnp.arange(HH) % HEADS * HID + jnp.arange(HH) // HEADS
    hpA, hpB, sa = _tc_in(x_p, W_in, b_in, W0[:, p], _asad_mat(as0, ad0)[p])
    oA, oB = edge_phase(hpA, hpB, sa)
    hpA, hpB, sa = _tc_mid(oA, oB, bias0[p], W1[p][:, p], _asad_mat(as1, ad1)[p])
    oA, oB = edge_phase(hpA, hpB, sa)
    hpA, hpB, sa = _tc_mid(oA, oB, bias1[p], W2[p][:, p], _asad_mat(as2, ad2)[p])
    oA, oB = edge_phase(hpA, hpB, sa)

    ssum = jnp.repeat(jnp.eye(HID, dtype=jnp.float32), HEADS, axis=0)
    return _tc_fin(oA, oB, ssum[:256], ssum[256:], bias2, Wo1, bo1, Wo2, bo2,
                   Wo3, bo3)
